# baseline (speedup 1.0000x reference)
import functools

import jax
import jax.numpy as jnp
from jax import lax
from jax.experimental import pallas as pl
from jax.experimental.pallas import tpu as pltpu
from jax.experimental.pallas import tpu_sc as plsc

_NE = 50000
_NA = 10000
_D = 128

_EP_EE = 262144
_EP_AE = 131072
_EP_EA = 131072

_LC_E = 50176
_LC_A = 10240

_NB = 4
_LAG = 2


def _pad_edges(ei, e_pad, dummy_dst):
    src = jnp.pad(ei[0], (0, e_pad - ei.shape[1]))
    dst = jnp.pad(ei[1], (0, e_pad - ei.shape[1]), constant_values=dummy_dst)
    return src.reshape(e_pad // 128, 128), dst.reshape(e_pad // 128, 128)


def _counts_body(dee_hbm, dae_hbm, dea_hbm, oee, oae, oea,
                 cee, cae, cea, dchunk):
    c = lax.axis_index("c")
    s = lax.axis_index("s")
    tid = c * 16 + s

    ones = jnp.ones((16,), jnp.float32)
    zeros = jnp.zeros((16,), jnp.float32)

    def _z(ref, n):
        def body(i, _):
            ref[pl.ds(i * 16, 16)] = zeros
            return 0
        lax.fori_loop(0, n // 16, body, 0)
    _z(cee, _LC_E)
    _z(cae, _LC_A)
    _z(cea, _LC_A)

    def _hist(dst_hbm, cnt_ref, rows_per_tile):
        def ic_body(ic, _):
            r0 = tid * rows_per_tile + ic * 16
            pltpu.sync_copy(dst_hbm.at[pl.ds(r0, 16), :], dchunk)

            def row_body(j, _):
                for k in range(8):
                    d16 = dchunk[j, pl.ds(k * 16, 16)]
                    plsc.addupdate_scatter(cnt_ref, [d16], ones)
                return 0
            lax.fori_loop(0, 16, row_body, 0)
            return 0
        lax.fori_loop(0, rows_per_tile // 16, ic_body, 0)

    _hist(dee_hbm, cee, _EP_EE // 128 // 32)
    _hist(dae_hbm, cae, _EP_AE // 128 // 32)
    _hist(dea_hbm, cea, _EP_EA // 128 // 32)

    pltpu.sync_copy(cee, oee.at[tid])
    pltpu.sync_copy(cae, oae.at[tid])
    pltpu.sync_copy(cea, oea.at[tid])


def _sc_counts(dst2_ee, dst2_ae, dst2_ea):
    mesh = plsc.VectorSubcoreMesh(core_axis_name="c", subcore_axis_name="s")
    out_type = (
        jax.ShapeDtypeStruct((32, _LC_E), jnp.float32),
        jax.ShapeDtypeStruct((32, _LC_A), jnp.float32),
        jax.ShapeDtypeStruct((32, _LC_A), jnp.float32),
    )
    scratch = [
        pltpu.VMEM((_LC_E,), jnp.float32),
        pltpu.VMEM((_LC_A,), jnp.float32),
        pltpu.VMEM((_LC_A,), jnp.float32),
        pltpu.VMEM((16, 128), jnp.int32),
    ]
    f = pl.kernel(_counts_body, out_type=out_type, mesh=mesh,
                  scratch_types=scratch,
                  compiler_params=pltpu.CompilerParams(
                      use_tc_tiling_on_sc=False, needs_layout_passes=False))
    return f(dst2_ee, dst2_ae, dst2_ea)


def _inv_body(p_ref, o_ref):
    s = jnp.sum(p_ref[...], axis=0)
    o_ref[...] = 1.0 / jnp.maximum(s, 1.0)


def _tc_invcnt(p, n):
    rows = p.shape[1] // 128
    out = pl.pallas_call(
        _inv_body,
        out_shape=jax.ShapeDtypeStruct((rows, 128), jnp.float32),
    )(p.reshape(32, rows, 128))
    return out.reshape(rows * 128, 1)[:n]


def _run_pass(tbl, src2, dst2, acc, sbuf, dbuf, rows, isems, gsems, ssems,
              s, n_ic):
    base_row = s * (n_ic * 16)
    pltpu.async_copy(src2.at[pl.ds(base_row, 16), :], sbuf.at[0], isems[0])
    pltpu.async_copy(dst2.at[pl.ds(base_row, 16), :], dbuf.at[0], isems[1])

    def ic_body(ic, _):
        sl = lax.rem(ic, 2)
        pltpu.make_async_copy(src2.at[pl.ds(base_row + ic * 16, 16), :],
                              sbuf.at[sl], isems[0]).wait()
        pltpu.make_async_copy(dst2.at[pl.ds(base_row + ic * 16, 16), :],
                              dbuf.at[sl], isems[1]).wait()

        @pl.when(ic + 1 < n_ic)
        def _():
            nsl = lax.rem(ic + 1, 2)
            nr = base_row + (ic + 1) * 16
            pltpu.async_copy(src2.at[pl.ds(nr, 16), :], sbuf.at[nsl],
                             isems[0])
            pltpu.async_copy(dst2.at[pl.ds(nr, 16), :], dbuf.at[nsl],
                             isems[1])

        gd = [None] * _NB
        sd = [None] * _NB
        for t in range(16 + _LAG):
            if t < 16:
                b = t % _NB
                if t >= _NB:
                    sd[b].wait()
                gd[b] = pltpu.async_copy(
                    tbl.at[sbuf.at[sl, t]],
                    rows.at[pl.ds(b * 128, 128), :], gsems[b])
            if t >= _LAG:
                i = t - _LAG
                bi = i % _NB
                gd[bi].wait()
                sd[bi] = pltpu.async_copy(
                    rows.at[pl.ds(bi * 128, 128), :],
                    acc.at[dbuf.at[sl, i]], ssems[bi], add=True)
        for i in range(16 - _NB, 16):
            sd[i % _NB].wait()
        return 0

    lax.fori_loop(0, n_ic, ic_body, 0)


def _layer_body(yee, yea, yae, se_ee, sd_ee, se_ea, sd_ea, se_ae, sd_ae,
                o_ee, o_ea, o_ae, acc, sbuf, dbuf, rows, zbuf,
                isems, gsems, ssems):
    c = lax.axis_index("c")
    s = lax.axis_index("s")
    zr = zbuf.shape[0]

    def zb(i, _):
        for k in range(2):
            zbuf[i, pl.ds(k * 16, 16)] = jnp.zeros((16,), jnp.float32)
        return 0
    lax.fori_loop(0, zr, zb, 0)

    phases = [
        (yee, se_ee, sd_ee, o_ee, _NE, 8),
        (yea, se_ea, sd_ea, o_ea, _NA, 4),
        (yae, se_ae, sd_ae, o_ae, _NA, 4),
    ]
    first = True
    for ytab, src2, dst2, out, n_dst, n_ic in phases:
        stripe = n_dst // 16
        for p in range(2):
            for z in range(stripe // zr):
                pltpu.sync_copy(zbuf,
                                acc.at[pl.ds(s * stripe + z * zr, zr), :])
            plsc.subcore_barrier()
            q = c * 2 + p
            _run_pass(ytab.at[q], src2, dst2, acc, sbuf, dbuf, rows,
                      isems, gsems, ssems, s, n_ic)
            plsc.subcore_barrier()
            pltpu.sync_copy(acc.at[pl.ds(s * stripe, stripe), :],
                            out.at[q, pl.ds(s * stripe, stripe), :])
        first = False


def _sc_layer(yee, yea, yae, se_ee, sd_ee, se_ea, sd_ea, se_ae, sd_ae):
    mesh = plsc.VectorSubcoreMesh(core_axis_name="c", subcore_axis_name="s")
    out_type = (
        jax.ShapeDtypeStruct((4, _NE, 32), jnp.float32),
        jax.ShapeDtypeStruct((4, _NA, 32), jnp.float32),
        jax.ShapeDtypeStruct((4, _NA, 32), jnp.float32),
    )
    scratch = [
        pltpu.VMEM_SHARED((_NE + 8, 32), jnp.float32),
        pltpu.VMEM((2, 16, 128), jnp.int32),
        pltpu.VMEM((2, 16, 128), jnp.int32),
        pltpu.VMEM((_NB * 128, 32), jnp.float32),
        pltpu.VMEM((125, 32), jnp.float32),
        [pltpu.SemaphoreType.DMA] * 2,
        [pltpu.SemaphoreType.DMA] * _NB,
        [pltpu.SemaphoreType.DMA] * _NB,
    ]
    f = pl.kernel(_layer_body, out_type=out_type, mesh=mesh,
                  scratch_types=scratch,
                  compiler_params=pltpu.CompilerParams(
                      use_tc_tiling_on_sc=False, needs_layout_passes=False))
    return f(yee, yea, yae, se_ee, sd_ee, se_ea, sd_ea, se_ae, sd_ae)


_BN = 1000


def _split_writes(cat, out_refs, specs):
    col = 0
    for o_ref, (nc, cc) in zip(out_refs, specs):
        for p in range(nc):
            o_ref[p] = cat[:, col:col + cc]
            col += cc


def _proj_cat_body(x_ref, w1_ref, b1_ref, wc_ref, *out_refs, specs):
    h = jnp.maximum(
        jnp.dot(x_ref[...], w1_ref[...],
                preferred_element_type=jnp.float32) + b1_ref[...], 0.0)
    cat = jnp.dot(h, wc_ref[...], preferred_element_type=jnp.float32)
    _split_writes(cat, out_refs, specs)


def _tc_proj_cat(x, w1, b1, wc, specs):
    n = x.shape[0]
    kcols = wc.shape[1]
    out_shape = [jax.ShapeDtypeStruct((nc, n, cc), jnp.float32)
                 for nc, cc in specs]
    out_specs = [pl.BlockSpec((nc, _BN, cc), lambda i: (0, i, 0))
                 for nc, cc in specs]
    return pl.pallas_call(
        functools.partial(_proj_cat_body, specs=specs),
        grid=(n // _BN,),
        in_specs=[
            pl.BlockSpec((_BN, _D), lambda i: (i, 0)),
            pl.BlockSpec((_D, _D), lambda i: (0, 0)),
            pl.BlockSpec((1, _D), lambda i: (0, 0)),
            pl.BlockSpec((_D, kcols), lambda i: (0, 0)),
        ],
        out_specs=out_specs,
        out_shape=out_shape,
    )(x, w1, b1, wc)


def _combine_cat_body(*refs, specs, has_ee, nb_a, final):
    it = iter(refs)
    m = jnp.zeros((_BN, _D), jnp.float32)
    if has_ee:
        see_ref = next(it)
        cee_ref = next(it)
        see = jnp.concatenate([see_ref[p] for p in range(4)], axis=-1)
        m = m + see * cee_ref[...]
    sa_ref = next(it)
    ca_ref = next(it)
    sa = jnp.concatenate([sa_ref[p] for p in range(4)], axis=-1)
    ma = sa * ca_ref[...]
    if has_ee:
        i = pl.program_id(0)
        ma = jnp.where(i < nb_a, ma, 0.0)
    m = m + ma
    r_ref = next(it)
    b_ref = next(it)
    wc_ref = next(it)
    h = jnp.maximum(m + r_ref[...] + b_ref[...], 0.0)
    cat = jnp.dot(h, wc_ref[...], preferred_element_type=jnp.float32)
    rest = list(it)
    if final:
        cat = cat + rest[0][...]
        rest = rest[1:]
    _split_writes(cat, rest, specs)


def _tc_combine_cat(s_ee, cnt_ee, s_a, cnt_a, r, b, wc, bo, specs):
    if r.ndim == 3:
        r = r[0]
    n = r.shape[0]
    has_ee = s_ee is not None
    nb_a = s_a.shape[1] // _BN
    kcols = wc.shape[1]
    in_specs = []
    args = []
    if has_ee:
        in_specs += [
            pl.BlockSpec((4, _BN, 32), lambda i: (0, i, 0)),
            pl.BlockSpec((_BN, 1), lambda i: (i, 0)),
        ]
        args += [s_ee, cnt_ee]
    cl = nb_a - 1
    in_specs += [
        pl.BlockSpec((4, _BN, 32),
                     lambda i, cl=cl: (0, jnp.minimum(i, cl), 0)),
        pl.BlockSpec((_BN, 1), lambda i, cl=cl: (jnp.minimum(i, cl), 0)),
        pl.BlockSpec((_BN, _D), lambda i: (i, 0)),
        pl.BlockSpec((1, _D), lambda i: (0, 0)),
        pl.BlockSpec((_D, kcols), lambda i: (0, 0)),
    ]
    args += [s_a, cnt_a, r, b, wc]
    if bo is not None:
        in_specs.append(pl.BlockSpec((1, kcols), lambda i: (0, 0)))
        args.append(bo)
    out_shape = [jax.ShapeDtypeStruct((nc, n, cc), jnp.float32)
                 for nc, cc in specs]
    out_specs = [pl.BlockSpec((nc, _BN, cc), lambda i: (0, i, 0))
                 for nc, cc in specs]
    body = functools.partial(_combine_cat_body, specs=specs, has_ee=has_ee,
                             nb_a=nb_a, final=bo is not None)
    return pl.pallas_call(
        body, grid=(n // _BN,), in_specs=in_specs, out_specs=out_specs,
        out_shape=out_shape,
    )(*args)


_E_SPECS = [(4, 32), (4, 32), (1, 128)]
_A_SPECS = [(4, 32), (1, 128)]
_Z_SPECS = [(1, 128)]


def kernel(x_entity, x_attribute, params, edge_index_ee, edge_index_ae,
           edge_index_ea):
    We, be = params['lin']['entity']
    Wa, ba = params['lin']['attribute']
    Woe, boe = params['out']['entity']
    Woa, boa = params['out']['attribute']
    convs = params['convs']

    src_ee, dst_ee = _pad_edges(edge_index_ee, _EP_EE, _NE)
    src_ae, dst_ae = _pad_edges(edge_index_ae, _EP_AE, _NA)
    src_ea, dst_ea = _pad_edges(edge_index_ea, _EP_EA, _NA)

    cnt_ee_p, cnt_ae_p, cnt_ea_p = _sc_counts(dst_ee, dst_ae, dst_ea)
    cnt_ee_p = _tc_invcnt(cnt_ee_p, _NE)
    cnt_ae_p = _tc_invcnt(cnt_ae_p, _NA)
    cnt_ea_p = _tc_invcnt(cnt_ea_p, _NA)

    def ewc(layer):
        Wl_ee, _, Wr_ee = layer['ee']
        Wl_ea, _, _ = layer['ea']
        _, _, Wr_ae = layer['ae']
        return jnp.concatenate([Wl_ee, Wl_ea, Wr_ee + Wr_ae], axis=1)

    def awc(layer):
        Wl_ae, _, _ = layer['ae']
        _, _, Wr_ea = layer['ea']
        return jnp.concatenate([Wl_ae, Wr_ea], axis=1)

    def ebias(layer):
        return (layer['ee'][1] + layer['ae'][1]).reshape(1, _D)

    def abias(layer):
        return layer['ea'][1].reshape(1, _D)

    yee, yea, r_e = _tc_proj_cat(x_entity, We, be.reshape(1, _D),
                                 ewc(convs[0]), _E_SPECS)
    yae, r_a = _tc_proj_cat(x_attribute, Wa, ba.reshape(1, _D),
                            awc(convs[0]), _A_SPECS)

    for li in range(2):
        s_ee, s_ea, s_ae = _sc_layer(yee, yea, yae, src_ee, dst_ee,
                                     src_ea, dst_ea, src_ae, dst_ae)
        if li == 0:
            yee, yea, r_e = _tc_combine_cat(
                s_ee, cnt_ee_p, s_ae, cnt_ae_p, r_e, ebias(convs[0]),
                ewc(convs[1]), None, _E_SPECS)
            yae, r_a = _tc_combine_cat(
                None, None, s_ea, cnt_ea_p, r_a, abias(convs[0]),
                awc(convs[1]), None, _A_SPECS)
        else:
            (z_e,) = _tc_combine_cat(
                s_ee, cnt_ee_p, s_ae, cnt_ae_p, r_e, ebias(convs[1]),
                Woe, boe.reshape(1, _D), _Z_SPECS)
            (z_a,) = _tc_combine_cat(
                None, None, s_ea, cnt_ea_p, r_a, abias(convs[1]),
                Woa, boa.reshape(1, _D), _Z_SPECS)

    return (z_e[0], z_a[0])


# speedup vs baseline: 1.1403x; 1.1403x over previous
import functools

import jax
import jax.numpy as jnp
from jax import lax
from jax.experimental import pallas as pl
from jax.experimental.pallas import tpu as pltpu
from jax.experimental.pallas import tpu_sc as plsc

_NE = 50000
_NA = 10000
_D = 128

_EP_EE = 262144
_EP_AE = 131072
_EP_EA = 131072

_LC_E = 50176
_LC_A = 10240

_NB = 4
_LAG = 2


def _pad_edges(ei, e_pad, dummy_dst):
    src = jnp.pad(ei[0], (0, e_pad - ei.shape[1]))
    dst = jnp.pad(ei[1], (0, e_pad - ei.shape[1]), constant_values=dummy_dst)
    return src.reshape(e_pad // 128, 128), dst.reshape(e_pad // 128, 128)


def _counts_body(dee_hbm, dae_hbm, dea_hbm, oee, oae, oea,
                 cee, cae, cea, dchunk):
    c = lax.axis_index("c")
    s = lax.axis_index("s")
    tid = c * 16 + s

    ones = jnp.ones((16,), jnp.float32)
    zeros = jnp.zeros((16,), jnp.float32)

    def _z(ref, n):
        def body(i, _):
            ref[pl.ds(i * 16, 16)] = zeros
            return 0
        lax.fori_loop(0, n // 16, body, 0)
    _z(cee, _LC_E)
    _z(cae, _LC_A)
    _z(cea, _LC_A)

    def _hist(dst_hbm, cnt_ref, rows_per_tile):
        def ic_body(ic, _):
            r0 = tid * rows_per_tile + ic * 16
            pltpu.sync_copy(dst_hbm.at[pl.ds(r0, 16), :], dchunk)

            def row_body(j, _):
                for k in range(8):
                    d16 = dchunk[j, pl.ds(k * 16, 16)]
                    plsc.addupdate_scatter(cnt_ref, [d16], ones)
                return 0
            lax.fori_loop(0, 16, row_body, 0)
            return 0
        lax.fori_loop(0, rows_per_tile // 16, ic_body, 0)

    _hist(dee_hbm, cee, _EP_EE // 128 // 32)
    _hist(dae_hbm, cae, _EP_AE // 128 // 32)
    _hist(dea_hbm, cea, _EP_EA // 128 // 32)

    pltpu.sync_copy(cee, oee.at[tid])
    pltpu.sync_copy(cae, oae.at[tid])
    pltpu.sync_copy(cea, oea.at[tid])


def _sc_counts(dst2_ee, dst2_ae, dst2_ea):
    mesh = plsc.VectorSubcoreMesh(core_axis_name="c", subcore_axis_name="s")
    out_type = (
        jax.ShapeDtypeStruct((32, _LC_E), jnp.float32),
        jax.ShapeDtypeStruct((32, _LC_A), jnp.float32),
        jax.ShapeDtypeStruct((32, _LC_A), jnp.float32),
    )
    scratch = [
        pltpu.VMEM((_LC_E,), jnp.float32),
        pltpu.VMEM((_LC_A,), jnp.float32),
        pltpu.VMEM((_LC_A,), jnp.float32),
        pltpu.VMEM((16, 128), jnp.int32),
    ]
    f = pl.kernel(_counts_body, out_type=out_type, mesh=mesh,
                  scratch_types=scratch,
                  compiler_params=pltpu.CompilerParams(
                      use_tc_tiling_on_sc=False, needs_layout_passes=False))
    return f(dst2_ee, dst2_ae, dst2_ea)


def _inv_body(p_ref, o_ref):
    s = jnp.sum(p_ref[...], axis=0)
    o_ref[...] = 1.0 / jnp.maximum(s, 1.0)


def _tc_invcnt(p, n):
    rows = p.shape[1] // 128
    out = pl.pallas_call(
        _inv_body,
        out_shape=jax.ShapeDtypeStruct((rows, 128), jnp.float32),
    )(p.reshape(32, rows, 128))
    return out.reshape(rows * 128, 1)[:n]


def _do_ic(tbl, acc, sbuf, dbuf, rows, gsems, ssems, sl):
    gd = [None] * _NB
    sd = [None] * _NB
    for t in range(16 + _LAG):
        if t < 16:
            b = t % _NB
            if t >= _NB:
                sd[b].wait()
            gd[b] = pltpu.async_copy(
                tbl.at[sbuf.at[sl, t]],
                rows.at[pl.ds(b * 128, 128), :], gsems[b])
        if t >= _LAG:
            i = t - _LAG
            bi = i % _NB
            gd[bi].wait()
            sd[bi] = pltpu.async_copy(
                rows.at[pl.ds(bi * 128, 128), :],
                acc.at[dbuf.at[sl, i]], ssems[bi], add=True)
    for i in range(16 - _NB, 16):
        sd[i % _NB].wait()


def _run_pass(tbl, src2, dst2, acc, sbuf, dbuf, rows, isems, gsems, ssems,
              s, n_ic):
    base = s * (n_ic * 16)

    def fetch(row, sl):
        pltpu.async_copy(src2.at[pl.ds(row, 16), :], sbuf.at[sl], isems[0])
        pltpu.async_copy(dst2.at[pl.ds(row, 16), :], dbuf.at[sl], isems[1])

    def wait_fetch(row, sl):
        pltpu.make_async_copy(src2.at[pl.ds(row, 16), :], sbuf.at[sl],
                              isems[0]).wait()
        pltpu.make_async_copy(dst2.at[pl.ds(row, 16), :], dbuf.at[sl],
                              isems[1]).wait()

    fetch(base, 0)

    def pair_body(j, _):
        r0 = base + 2 * j * 16
        wait_fetch(r0, 0)
        fetch(r0 + 16, 1)
        _do_ic(tbl, acc, sbuf, dbuf, rows, gsems, ssems, 0)
        wait_fetch(r0 + 16, 1)

        @pl.when(2 * j + 2 < n_ic)
        def _():
            fetch(r0 + 32, 0)

        _do_ic(tbl, acc, sbuf, dbuf, rows, gsems, ssems, 1)
        return 0

    lax.fori_loop(0, n_ic // 2, pair_body, 0)


def _seg_phases(phases, acc, sbuf, dbuf, rows, zbuf, isems, gsems, ssems):
    c = lax.axis_index("c")
    s = lax.axis_index("s")
    zr = zbuf.shape[0]
    dc = zbuf.shape[1]

    def zb(i, _):
        for k in range(dc // 16):
            zbuf[i, pl.ds(k * 16, 16)] = jnp.zeros((16,), jnp.float32)
        return 0
    lax.fori_loop(0, zr, zb, 0)

    for ytab, src2, dst2, out, n_dst, n_ic, n_pass in phases:
        stripe = n_dst // 16
        for p in range(n_pass):
            for z in range(stripe // zr):
                pltpu.sync_copy(zbuf,
                                acc.at[pl.ds(s * stripe + z * zr, zr), :])
            plsc.subcore_barrier()
            q = c * n_pass + p
            _run_pass(ytab.at[q], src2, dst2, acc, sbuf, dbuf, rows,
                      isems, gsems, ssems, s, n_ic)
            plsc.subcore_barrier()
            pltpu.sync_copy(acc.at[pl.ds(s * stripe, stripe), :],
                            out.at[q, pl.ds(s * stripe, stripe), :])


def _ee_body(yee, se, sd, out, acc, sbuf, dbuf, rows, zbuf,
             isems, gsems, ssems):
    _seg_phases([(yee, se, sd, out, _NE, 8, 2)],
                acc, sbuf, dbuf, rows, zbuf, isems, gsems, ssems)


def _a_body(yea, yae, se_ea, sd_ea, se_ae, sd_ae, o_ea, o_ae,
            acc, sbuf, dbuf, rows, zbuf, isems, gsems, ssems):
    _seg_phases([(yea, se_ea, sd_ea, o_ea, _NA, 4, 1),
                 (yae, se_ae, sd_ae, o_ae, _NA, 4, 1)],
                acc, sbuf, dbuf, rows, zbuf, isems, gsems, ssems)


def _sc_scratch(n_dst, dc):
    return [
        pltpu.VMEM_SHARED((n_dst + 8, dc), jnp.float32),
        pltpu.VMEM((2, 16, 128), jnp.int32),
        pltpu.VMEM((2, 16, 128), jnp.int32),
        pltpu.VMEM((_NB * 128, dc), jnp.float32),
        pltpu.VMEM((125, dc), jnp.float32),
        [pltpu.SemaphoreType.DMA] * 2,
        [pltpu.SemaphoreType.DMA] * _NB,
        [pltpu.SemaphoreType.DMA] * _NB,
    ]


_SC_PARAMS = pltpu.CompilerParams(use_tc_tiling_on_sc=False,
                                  needs_layout_passes=False)
_SC_MESH = dict(core_axis_name="c", subcore_axis_name="s")


def _sc_ee(yee, se, sd):
    f = pl.kernel(_ee_body,
                  out_type=jax.ShapeDtypeStruct((4, _NE, 32), jnp.float32),
                  mesh=plsc.VectorSubcoreMesh(**_SC_MESH),
                  scratch_types=_sc_scratch(_NE, 32),
                  compiler_params=_SC_PARAMS)
    return f(yee, se, sd)


def _sc_a(yea, yae, se_ea, sd_ea, se_ae, sd_ae):
    out_type = (
        jax.ShapeDtypeStruct((2, _NA, 64), jnp.float32),
        jax.ShapeDtypeStruct((2, _NA, 64), jnp.float32),
    )
    f = pl.kernel(_a_body, out_type=out_type,
                  mesh=plsc.VectorSubcoreMesh(**_SC_MESH),
                  scratch_types=_sc_scratch(_NA, 64),
                  compiler_params=_SC_PARAMS)
    return f(yea, yae, se_ea, sd_ea, se_ae, sd_ae)


_BN = 1000


def _split_writes(cat, out_refs, specs):
    col = 0
    for o_ref, (nc, cc) in zip(out_refs, specs):
        for p in range(nc):
            o_ref[p] = cat[:, col:col + cc]
            col += cc


def _proj_cat_body(x_ref, w1_ref, b1_ref, wc_ref, *out_refs, specs):
    h = jnp.maximum(
        jnp.dot(x_ref[...], w1_ref[...],
                preferred_element_type=jnp.float32) + b1_ref[...], 0.0)
    cat = jnp.dot(h, wc_ref[...], preferred_element_type=jnp.float32)
    _split_writes(cat, out_refs, specs)


def _tc_proj_cat(x, w1, b1, wc, specs):
    n = x.shape[0]
    kcols = wc.shape[1]
    out_shape = [jax.ShapeDtypeStruct((nc, n, cc), jnp.float32)
                 for nc, cc in specs]
    out_specs = [pl.BlockSpec((nc, _BN, cc), lambda i: (0, i, 0))
                 for nc, cc in specs]
    return pl.pallas_call(
        functools.partial(_proj_cat_body, specs=specs),
        grid=(n // _BN,),
        in_specs=[
            pl.BlockSpec((_BN, _D), lambda i: (i, 0)),
            pl.BlockSpec((_D, _D), lambda i: (0, 0)),
            pl.BlockSpec((1, _D), lambda i: (0, 0)),
            pl.BlockSpec((_D, kcols), lambda i: (0, 0)),
        ],
        out_specs=out_specs,
        out_shape=out_shape,
    )(x, w1, b1, wc)


def _combine_cat_body(*refs, specs, has_ee, nb_a, final):
    it = iter(refs)
    m = jnp.zeros((_BN, _D), jnp.float32)
    if has_ee:
        see_ref = next(it)
        cee_ref = next(it)
        see = jnp.concatenate([see_ref[p] for p in range(4)], axis=-1)
        m = m + see * cee_ref[...]
    sa_ref = next(it)
    ca_ref = next(it)
    sa = jnp.concatenate([sa_ref[0], sa_ref[1]], axis=-1)
    ma = sa * ca_ref[...]
    if has_ee:
        i = pl.program_id(0)
        ma = jnp.where(i < nb_a, ma, 0.0)
    m = m + ma
    r_ref = next(it)
    b_ref = next(it)
    wc_ref = next(it)
    h = jnp.maximum(m + r_ref[...] + b_ref[...], 0.0)
    cat = jnp.dot(h, wc_ref[...], preferred_element_type=jnp.float32)
    rest = list(it)
    if final:
        cat = cat + rest[0][...]
        rest = rest[1:]
    _split_writes(cat, rest, specs)


def _tc_combine_cat(s_ee, cnt_ee, s_a, cnt_a, r, b, wc, bo, specs):
    if r.ndim == 3:
        r = r[0]
    n = r.shape[0]
    has_ee = s_ee is not None
    nb_a = s_a.shape[1] // _BN
    kcols = wc.shape[1]
    in_specs = []
    args = []
    if has_ee:
        in_specs += [
            pl.BlockSpec((4, _BN, 32), lambda i: (0, i, 0)),
            pl.BlockSpec((_BN, 1), lambda i: (i, 0)),
        ]
        args += [s_ee, cnt_ee]
    cl = nb_a - 1
    in_specs += [
        pl.BlockSpec((2, _BN, 64),
                     lambda i, cl=cl: (0, jnp.minimum(i, cl), 0)),
        pl.BlockSpec((_BN, 1), lambda i, cl=cl: (jnp.minimum(i, cl), 0)),
        pl.BlockSpec((_BN, _D), lambda i: (i, 0)),
        pl.BlockSpec((1, _D), lambda i: (0, 0)),
        pl.BlockSpec((_D, kcols), lambda i: (0, 0)),
    ]
    args += [s_a, cnt_a, r, b, wc]
    if bo is not None:
        in_specs.append(pl.BlockSpec((1, kcols), lambda i: (0, 0)))
        args.append(bo)
    out_shape = [jax.ShapeDtypeStruct((nc, n, cc), jnp.float32)
                 for nc, cc in specs]
    out_specs = [pl.BlockSpec((nc, _BN, cc), lambda i: (0, i, 0))
                 for nc, cc in specs]
    body = functools.partial(_combine_cat_body, specs=specs, has_ee=has_ee,
                             nb_a=nb_a, final=bo is not None)
    return pl.pallas_call(
        body, grid=(n // _BN,), in_specs=in_specs, out_specs=out_specs,
        out_shape=out_shape,
    )(*args)


_E_SPECS = [(4, 32), (2, 64), (1, 128)]
_A_SPECS = [(2, 64), (1, 128)]
_Z_SPECS = [(1, 128)]


def kernel(x_entity, x_attribute, params, edge_index_ee, edge_index_ae,
           edge_index_ea):
    We, be = params['lin']['entity']
    Wa, ba = params['lin']['attribute']
    Woe, boe = params['out']['entity']
    Woa, boa = params['out']['attribute']
    convs = params['convs']

    src_ee, dst_ee = _pad_edges(edge_index_ee, _EP_EE, _NE)
    src_ae, dst_ae = _pad_edges(edge_index_ae, _EP_AE, _NA)
    src_ea, dst_ea = _pad_edges(edge_index_ea, _EP_EA, _NA)

    cnt_ee_p, cnt_ae_p, cnt_ea_p = _sc_counts(dst_ee, dst_ae, dst_ea)
    cnt_ee_p = _tc_invcnt(cnt_ee_p, _NE)
    cnt_ae_p = _tc_invcnt(cnt_ae_p, _NA)
    cnt_ea_p = _tc_invcnt(cnt_ea_p, _NA)

    def ewc(layer):
        Wl_ee, _, Wr_ee = layer['ee']
        Wl_ea, _, _ = layer['ea']
        _, _, Wr_ae = layer['ae']
        return jnp.concatenate([Wl_ee, Wl_ea, Wr_ee + Wr_ae], axis=1)

    def awc(layer):
        Wl_ae, _, _ = layer['ae']
        _, _, Wr_ea = layer['ea']
        return jnp.concatenate([Wl_ae, Wr_ea], axis=1)

    def ebias(layer):
        return (layer['ee'][1] + layer['ae'][1]).reshape(1, _D)

    def abias(layer):
        return layer['ea'][1].reshape(1, _D)

    yee, yea, r_e = _tc_proj_cat(x_entity, We, be.reshape(1, _D),
                                 ewc(convs[0]), _E_SPECS)
    yae, r_a = _tc_proj_cat(x_attribute, Wa, ba.reshape(1, _D),
                            awc(convs[0]), _A_SPECS)

    for li in range(2):
        s_ee = _sc_ee(yee, src_ee, dst_ee)
        s_ea, s_ae = _sc_a(yea, yae, src_ea, dst_ea, src_ae, dst_ae)
        if li == 0:
            yee, yea, r_e = _tc_combine_cat(
                s_ee, cnt_ee_p, s_ae, cnt_ae_p, r_e, ebias(convs[0]),
                ewc(convs[1]), None, _E_SPECS)
            yae, r_a = _tc_combine_cat(
                None, None, s_ea, cnt_ea_p, r_a, abias(convs[0]),
                awc(convs[1]), None, _A_SPECS)
        else:
            (z_e,) = _tc_combine_cat(
                s_ee, cnt_ee_p, s_ae, cnt_ae_p, r_e, ebias(convs[1]),
                Woe, boe.reshape(1, _D), _Z_SPECS)
            (z_a,) = _tc_combine_cat(
                None, None, s_ea, cnt_ea_p, r_a, abias(convs[1]),
                Woa, boa.reshape(1, _D), _Z_SPECS)

    return (z_e[0], z_a[0])


# speedup vs baseline: 1.2410x; 1.0883x over previous
import functools

import jax
import jax.numpy as jnp
from jax import lax
from jax.experimental import pallas as pl
from jax.experimental.pallas import tpu as pltpu
from jax.experimental.pallas import tpu_sc as plsc

_NE = 50000
_NA = 10000
_D = 128

_EP_EE = 262144
_EP_AE = 131072
_EP_EA = 131072

_LC_E = 50176
_LC_A = 10240

_NB = 4
_LAG = 2


def _pad_edges(ei, e_pad, dummy_dst, nch):
    src = jnp.pad(ei[0], (0, e_pad - ei.shape[1]))
    dst = jnp.pad(ei[1], (0, e_pad - ei.shape[1]), constant_values=dummy_dst)
    src2 = src.reshape(e_pad // 128, 128)
    sidx = src2[None] * nch + jnp.arange(nch, dtype=jnp.int32)[:, None, None]
    return sidx, dst.reshape(e_pad // 128, 128)


def _counts_body(dee_hbm, dae_hbm, dea_hbm, oee, oae, oea,
                 cee, cae, cea, dchunk):
    c = lax.axis_index("c")
    s = lax.axis_index("s")
    tid = c * 16 + s

    ones = jnp.ones((16,), jnp.float32)
    zeros = jnp.zeros((16,), jnp.float32)
    m127 = jnp.full((16,), 127, jnp.int32)

    def _z(ref):
        def body(i, _):
            for k in range(8):
                ref[i, pl.ds(k * 16, 16)] = zeros
            return 0
        lax.fori_loop(0, ref.shape[0], body, 0)
    _z(cee)
    _z(cae)
    _z(cea)

    def _hist(dst_hbm, cnt_ref, rows_per_tile):
        def ic_body(ic, _):
            r0 = tid * rows_per_tile + ic * 16
            pltpu.sync_copy(dst_hbm.at[pl.ds(r0, 16), :], dchunk)

            def row_body(j, _):
                for k in range(8):
                    d16 = dchunk[j, pl.ds(k * 16, 16)]
                    plsc.addupdate_scatter(
                        cnt_ref, [lax.shift_right_logical(d16, 7),
                                  lax.bitwise_and(d16, m127)], ones)
                return 0
            lax.fori_loop(0, 16, row_body, 0)
            return 0
        lax.fori_loop(0, rows_per_tile // 16, ic_body, 0)

    _hist(dee_hbm, cee, _EP_EE // 128 // 32)
    _hist(dae_hbm, cae, _EP_AE // 128 // 32)
    _hist(dea_hbm, cea, _EP_EA // 128 // 32)

    pltpu.sync_copy(cee, oee.at[tid])
    pltpu.sync_copy(cae, oae.at[tid])
    pltpu.sync_copy(cea, oea.at[tid])


def _sc_counts(dst2_ee, dst2_ae, dst2_ea):
    mesh = plsc.VectorSubcoreMesh(core_axis_name="c", subcore_axis_name="s")
    out_type = (
        jax.ShapeDtypeStruct((32, _LC_E // 128, 128), jnp.float32),
        jax.ShapeDtypeStruct((32, _LC_A // 128, 128), jnp.float32),
        jax.ShapeDtypeStruct((32, _LC_A // 128, 128), jnp.float32),
    )
    scratch = [
        pltpu.VMEM((_LC_E // 128, 128), jnp.float32),
        pltpu.VMEM((_LC_A // 128, 128), jnp.float32),
        pltpu.VMEM((_LC_A // 128, 128), jnp.float32),
        pltpu.VMEM((16, 128), jnp.int32),
    ]
    f = pl.kernel(_counts_body, out_type=out_type, mesh=mesh,
                  scratch_types=scratch,
                  compiler_params=pltpu.CompilerParams(
                      use_tc_tiling_on_sc=False, needs_layout_passes=False))
    return f(dst2_ee, dst2_ae, dst2_ea)


def _inv_body(p_ref, o_ref):
    s = jnp.sum(p_ref[...], axis=0)
    o_ref[...] = 1.0 / jnp.maximum(s, 1.0)


def _tc_invcnt(p, n):
    rows = p.shape[1]
    out = pl.pallas_call(
        _inv_body,
        out_shape=jax.ShapeDtypeStruct((rows, 128), jnp.float32),
    )(p)
    return out.reshape(rows * 128, 1)[:n]


def _do_ic(tbl, acc, sbuf, dbuf, rows, gsems, ssems, sl, cw):
    gd = [None] * _NB
    sd = [None] * _NB
    for t in range(16 + _LAG):
        if t < 16:
            b = t % _NB
            if t >= _NB:
                sd[b].wait()
            gd[b] = pltpu.async_copy(
                tbl.at[sbuf.at[sl, t]],
                rows.at[pl.ds(b * 128, 128), pl.ds(0, cw)], gsems[b])
        if t >= _LAG:
            i = t - _LAG
            bi = i % _NB
            gd[bi].wait()
            sd[bi] = pltpu.async_copy(
                rows.at[pl.ds(bi * 128, 128), pl.ds(0, cw)],
                acc.at[dbuf.at[sl, i]], ssems[bi], add=True)
    for i in range(16 - _NB, 16):
        sd[i % _NB].wait()


def _run_pass(tbl, src2, dst2, acc, sbuf, dbuf, rows, isems, gsems, ssems,
              s, n_ic, cw):
    base = s * (n_ic * 16)

    def fetch(row, sl):
        pltpu.async_copy(src2.at[pl.ds(row, 16), :], sbuf.at[sl], isems[0])
        pltpu.async_copy(dst2.at[pl.ds(row, 16), :], dbuf.at[sl], isems[1])

    def wait_fetch(row, sl):
        pltpu.make_async_copy(src2.at[pl.ds(row, 16), :], sbuf.at[sl],
                              isems[0]).wait()
        pltpu.make_async_copy(dst2.at[pl.ds(row, 16), :], dbuf.at[sl],
                              isems[1]).wait()

    fetch(base, 0)

    def pair_body(j, _):
        r0 = base + 2 * j * 16
        wait_fetch(r0, 0)
        fetch(r0 + 16, 1)
        _do_ic(tbl, acc, sbuf, dbuf, rows, gsems, ssems, 0, cw)
        wait_fetch(r0 + 16, 1)

        @pl.when(2 * j + 2 < n_ic)
        def _():
            fetch(r0 + 32, 0)

        _do_ic(tbl, acc, sbuf, dbuf, rows, gsems, ssems, 1, cw)
        return 0

    lax.fori_loop(0, n_ic // 2, pair_body, 0)


def _seg_phases(phases, acc, sbuf, dbuf, rows, zbuf, isems, gsems, ssems):
    c = lax.axis_index("c")
    s = lax.axis_index("s")
    zr = zbuf.shape[0]

    def zb(i, _):
        for k in range(zbuf.shape[1] // 16):
            zbuf[i, pl.ds(k * 16, 16)] = jnp.zeros((16,), jnp.float32)
        return 0
    lax.fori_loop(0, zr, zb, 0)

    for ytab, sidx, dst2, out, n_dst, n_ic, n_pass, cw in phases:
        stripe = n_dst // 16
        for p in range(n_pass):
            for z in range(stripe // zr):
                pltpu.sync_copy(
                    zbuf.at[:, pl.ds(0, cw)],
                    acc.at[pl.ds(s * stripe + z * zr, zr), pl.ds(0, cw)])
            plsc.subcore_barrier()
            q = c * n_pass + p
            _run_pass(ytab, sidx.at[q], dst2, acc, sbuf, dbuf, rows,
                      isems, gsems, ssems, s, n_ic, cw)
            plsc.subcore_barrier()
            pltpu.sync_copy(
                acc.at[pl.ds(s * stripe, stripe), pl.ds(0, cw)],
                out.at[pl.ds(s * stripe, stripe), pl.ds(cw * q, cw)])


def _ee_body(yee, sidx, sd, out, acc, sbuf, dbuf, rows, zbuf,
             isems, gsems, ssems):
    _seg_phases([(yee, sidx, sd, out, _NE, 8, 2, 32)],
                acc, sbuf, dbuf, rows, zbuf, isems, gsems, ssems)


def _a_body(yea, yae, si_ea, sd_ea, si_ae, sd_ae, o_ea, o_ae,
            acc, sbuf, dbuf, rows, zbuf, isems, gsems, ssems):
    _seg_phases([(yea, si_ea, sd_ea, o_ea, _NA, 4, 1, 64),
                 (yae, si_ae, sd_ae, o_ae, _NA, 4, 1, 64)],
                acc, sbuf, dbuf, rows, zbuf, isems, gsems, ssems)


def _sc_scratch(n_dst, dc, zr):
    return [
        pltpu.VMEM_SHARED((n_dst + 8, dc), jnp.float32),
        pltpu.VMEM((2, 16, 128), jnp.int32),
        pltpu.VMEM((2, 16, 128), jnp.int32),
        pltpu.VMEM((_NB * 128, dc), jnp.float32),
        pltpu.VMEM((zr, dc), jnp.float32),
        [pltpu.SemaphoreType.DMA] * 2,
        [pltpu.SemaphoreType.DMA] * _NB,
        [pltpu.SemaphoreType.DMA] * _NB,
    ]


_SC_PARAMS = pltpu.CompilerParams(use_tc_tiling_on_sc=False,
                                  needs_layout_passes=False)
_SC_MESH = dict(core_axis_name="c", subcore_axis_name="s")


def _sc_ee(yee, sidx, sd):
    f = pl.kernel(_ee_body,
                  out_type=jax.ShapeDtypeStruct((_NE, 128), jnp.float32),
                  mesh=plsc.VectorSubcoreMesh(**_SC_MESH),
                  scratch_types=_sc_scratch(_NE, 32, 125),
                  compiler_params=_SC_PARAMS)
    return f(yee.reshape(4 * _NE, 32), sidx, sd)


def _sc_a(yea, yae, si_ea, sd_ea, si_ae, sd_ae):
    out_type = (
        jax.ShapeDtypeStruct((_NA, 128), jnp.float32),
        jax.ShapeDtypeStruct((_NA, 128), jnp.float32),
    )
    f = pl.kernel(_a_body, out_type=out_type,
                  mesh=plsc.VectorSubcoreMesh(**_SC_MESH),
                  scratch_types=_sc_scratch(_NA, 64, 125),
                  compiler_params=_SC_PARAMS)
    return f(yea.reshape(2 * _NE, 64), yae.reshape(2 * _NA, 64),
             si_ea, sd_ea, si_ae, sd_ae)


_BN = 1000


def _split_writes(cat, out_refs, widths):
    col = 0
    for o_ref, w in zip(out_refs, widths):
        o_ref[...] = cat[:, col:col + w]
        col += w


def _proj_cat_body(x_ref, w1_ref, b1_ref, wc_ref, *out_refs, widths):
    h = jnp.maximum(
        jnp.dot(x_ref[...], w1_ref[...],
                preferred_element_type=jnp.float32) + b1_ref[...], 0.0)
    cat = jnp.dot(h, wc_ref[...], preferred_element_type=jnp.float32)
    _split_writes(cat, out_refs, widths)


def _tc_proj_cat(x, w1, b1, wc, widths):
    n = x.shape[0]
    kcols = wc.shape[1]
    out_shape = [jax.ShapeDtypeStruct((n, w), jnp.float32) for w in widths]
    out_specs = [pl.BlockSpec((_BN, w), lambda i: (i, 0)) for w in widths]
    return pl.pallas_call(
        functools.partial(_proj_cat_body, widths=widths),
        grid=(n // _BN,),
        in_specs=[
            pl.BlockSpec((_BN, _D), lambda i: (i, 0)),
            pl.BlockSpec((_D, _D), lambda i: (0, 0)),
            pl.BlockSpec((1, _D), lambda i: (0, 0)),
            pl.BlockSpec((_D, kcols), lambda i: (0, 0)),
        ],
        out_specs=out_specs,
        out_shape=out_shape,
    )(x, w1, b1, wc)


def _combine_cat_body(*refs, widths, has_ee, nb_a, final):
    it = iter(refs)
    m = jnp.zeros((_BN, _D), jnp.float32)
    if has_ee:
        m = m + next(it)[...] * next(it)[...]
    ma = next(it)[...] * next(it)[...]
    if has_ee:
        i = pl.program_id(0)
        ma = jnp.where(i < nb_a, ma, 0.0)
    m = m + ma
    r_ref = next(it)
    b_ref = next(it)
    wc_ref = next(it)
    h = jnp.maximum(m + r_ref[...] + b_ref[...], 0.0)
    cat = jnp.dot(h, wc_ref[...], preferred_element_type=jnp.float32)
    rest = list(it)
    if final:
        cat = cat + rest[0][...]
        rest = rest[1:]
    _split_writes(cat, rest, widths)


def _tc_combine_cat(s_ee, cnt_ee, s_a, cnt_a, r, b, wc, bo, widths):
    n = r.shape[0]
    has_ee = s_ee is not None
    nb_a = s_a.shape[0] // _BN
    kcols = wc.shape[1]
    in_specs = []
    args = []
    if has_ee:
        in_specs += [
            pl.BlockSpec((_BN, _D), lambda i: (i, 0)),
            pl.BlockSpec((_BN, 1), lambda i: (i, 0)),
        ]
        args += [s_ee, cnt_ee]
    cl = nb_a - 1
    in_specs += [
        pl.BlockSpec((_BN, _D), lambda i, cl=cl: (jnp.minimum(i, cl), 0)),
        pl.BlockSpec((_BN, 1), lambda i, cl=cl: (jnp.minimum(i, cl), 0)),
        pl.BlockSpec((_BN, _D), lambda i: (i, 0)),
        pl.BlockSpec((1, _D), lambda i: (0, 0)),
        pl.BlockSpec((_D, kcols), lambda i: (0, 0)),
    ]
    args += [s_a, cnt_a, r, b, wc]
    if bo is not None:
        in_specs.append(pl.BlockSpec((1, kcols), lambda i: (0, 0)))
        args.append(bo)
    out_shape = [jax.ShapeDtypeStruct((n, w), jnp.float32) for w in widths]
    out_specs = [pl.BlockSpec((_BN, w), lambda i: (i, 0)) for w in widths]
    body = functools.partial(_combine_cat_body, widths=widths, has_ee=has_ee,
                             nb_a=nb_a, final=bo is not None)
    return pl.pallas_call(
        body, grid=(n // _BN,), in_specs=in_specs, out_specs=out_specs,
        out_shape=out_shape,
    )(*args)


_E_WIDTHS = [128, 128, 128]
_A_WIDTHS = [128, 128]
_Z_WIDTHS = [128]


def kernel(x_entity, x_attribute, params, edge_index_ee, edge_index_ae,
           edge_index_ea):
    We, be = params['lin']['entity']
    Wa, ba = params['lin']['attribute']
    Woe, boe = params['out']['entity']
    Woa, boa = params['out']['attribute']
    convs = params['convs']

    si_ee, dst_ee = _pad_edges(edge_index_ee, _EP_EE, _NE, 4)
    si_ae, dst_ae = _pad_edges(edge_index_ae, _EP_AE, _NA, 2)
    si_ea, dst_ea = _pad_edges(edge_index_ea, _EP_EA, _NA, 2)

    cnt_ee_p, cnt_ae_p, cnt_ea_p = _sc_counts(dst_ee, dst_ae, dst_ea)
    inv_ee = _tc_invcnt(cnt_ee_p, _NE)
    inv_ae = _tc_invcnt(cnt_ae_p, _NA)
    inv_ea = _tc_invcnt(cnt_ea_p, _NA)

    def ewc(layer):
        Wl_ee, _, Wr_ee = layer['ee']
        Wl_ea, _, _ = layer['ea']
        _, _, Wr_ae = layer['ae']
        return jnp.concatenate([Wl_ee, Wl_ea, Wr_ee + Wr_ae], axis=1)

    def awc(layer):
        Wl_ae, _, _ = layer['ae']
        _, _, Wr_ea = layer['ea']
        return jnp.concatenate([Wl_ae, Wr_ea], axis=1)

    def ebias(layer):
        return (layer['ee'][1] + layer['ae'][1]).reshape(1, _D)

    def abias(layer):
        return layer['ea'][1].reshape(1, _D)

    yee, yea, r_e = _tc_proj_cat(x_entity, We, be.reshape(1, _D),
                                 ewc(convs[0]), _E_WIDTHS)
    yae, r_a = _tc_proj_cat(x_attribute, Wa, ba.reshape(1, _D),
                            awc(convs[0]), _A_WIDTHS)

    for li in range(2):
        s_ee = _sc_ee(yee, si_ee, dst_ee)
        s_ea, s_ae = _sc_a(yea, yae, si_ea, dst_ea, si_ae, dst_ae)
        if li == 0:
            yee, yea, r_e = _tc_combine_cat(
                s_ee, inv_ee, s_ae, inv_ae, r_e, ebias(convs[0]),
                ewc(convs[1]), None, _E_WIDTHS)
            yae, r_a = _tc_combine_cat(
                None, None, s_ea, inv_ea, r_a, abias(convs[0]),
                awc(convs[1]), None, _A_WIDTHS)
        else:
            (z_e,) = _tc_combine_cat(
                s_ee, inv_ee, s_ae, inv_ae, r_e, ebias(convs[1]),
                Woe, boe.reshape(1, _D), _Z_WIDTHS)
            (z_a,) = _tc_combine_cat(
                None, None, s_ea, inv_ea, r_a, abias(convs[1]),
                Woa, boa.reshape(1, _D), _Z_WIDTHS)

    return (z_e, z_a)


# speedup vs baseline: 1.4057x; 1.1327x over previous
import functools

import jax
import jax.numpy as jnp
from jax import lax
from jax.experimental import pallas as pl
from jax.experimental.pallas import tpu as pltpu
from jax.experimental.pallas import tpu_sc as plsc

_NE = 50000
_NA = 10000
_NEP = 51200
_NAP = 10240
_D = 128

_EP_EE = 262144
_EP_AE = 131072
_EP_EA = 131072

_LC_E = 51456
_LC_A = 10368

_NB = 4
_LAG = 2


def _pad_edges(ei, e_pad, dummy_dst):
    src = jnp.pad(ei[0], (0, e_pad - ei.shape[1]))
    dst = jnp.pad(ei[1], (0, e_pad - ei.shape[1]), constant_values=dummy_dst)
    return src.reshape(e_pad // 128, 128), dst.reshape(e_pad // 128, 128)


def _counts_body(dee_hbm, dae_hbm, dea_hbm, oee, oae, oea,
                 cee, cae, cea, dchunk):
    c = lax.axis_index("c")
    s = lax.axis_index("s")
    tid = c * 16 + s

    ones = jnp.ones((16,), jnp.float32)
    zeros = jnp.zeros((16,), jnp.float32)
    m127 = jnp.full((16,), 127, jnp.int32)

    def _z(ref):
        def body(i, _):
            for k in range(8):
                ref[i, pl.ds(k * 16, 16)] = zeros
            return 0
        lax.fori_loop(0, ref.shape[0], body, 0)
    _z(cee)
    _z(cae)
    _z(cea)

    def _hist(dst_hbm, cnt_ref, rows_per_tile):
        def ic_body(ic, _):
            r0 = tid * rows_per_tile + ic * 16
            pltpu.sync_copy(dst_hbm.at[pl.ds(r0, 16), :], dchunk)

            def row_body(j, _):
                for k in range(8):
                    d16 = dchunk[j, pl.ds(k * 16, 16)]
                    plsc.addupdate_scatter(
                        cnt_ref, [lax.shift_right_logical(d16, 7),
                                  lax.bitwise_and(d16, m127)], ones)
                return 0
            lax.fori_loop(0, 16, row_body, 0)
            return 0
        lax.fori_loop(0, rows_per_tile // 16, ic_body, 0)

    _hist(dee_hbm, cee, _EP_EE // 128 // 32)
    _hist(dae_hbm, cae, _EP_AE // 128 // 32)
    _hist(dea_hbm, cea, _EP_EA // 128 // 32)

    pltpu.sync_copy(cee, oee.at[tid])
    pltpu.sync_copy(cae, oae.at[tid])
    pltpu.sync_copy(cea, oea.at[tid])


def _sc_counts(dst2_ee, dst2_ae, dst2_ea):
    mesh = plsc.VectorSubcoreMesh(core_axis_name="c", subcore_axis_name="s")
    out_type = (
        jax.ShapeDtypeStruct((32, _LC_E // 128, 128), jnp.float32),
        jax.ShapeDtypeStruct((32, _LC_A // 128, 128), jnp.float32),
        jax.ShapeDtypeStruct((32, _LC_A // 128, 128), jnp.float32),
    )
    scratch = [
        pltpu.VMEM((_LC_E // 128, 128), jnp.float32),
        pltpu.VMEM((_LC_A // 128, 128), jnp.float32),
        pltpu.VMEM((_LC_A // 128, 128), jnp.float32),
        pltpu.VMEM((16, 128), jnp.int32),
    ]
    f = pl.kernel(_counts_body, out_type=out_type, mesh=mesh,
                  scratch_types=scratch,
                  compiler_params=pltpu.CompilerParams(
                      use_tc_tiling_on_sc=False, needs_layout_passes=False))
    return f(dst2_ee, dst2_ae, dst2_ea)


def _inv_body(p_ref, o_ref):
    s = jnp.sum(p_ref[...], axis=0)
    o_ref[...] = 1.0 / jnp.maximum(s, 1.0)


def _tc_invcnt(p, n):
    rows = p.shape[1]
    out = pl.pallas_call(
        _inv_body,
        out_shape=jax.ShapeDtypeStruct((rows, 128), jnp.float32),
    )(p)
    return out.reshape(rows * 128, 1)[:n]


def _do_ic(tbl, acc, sbuf, dbuf, rows, gsems, ssems, sl, cw):
    gd = [None] * _NB
    sd = [None] * _NB
    for t in range(16 + _LAG):
        if t < 16:
            b = t % _NB
            if t >= _NB:
                sd[b].wait()
            gd[b] = pltpu.async_copy(
                tbl.at[sbuf.at[sl, t]],
                rows.at[pl.ds(b * 128, 128), pl.ds(0, cw)], gsems[b])
        if t >= _LAG:
            i = t - _LAG
            bi = i % _NB
            gd[bi].wait()
            sd[bi] = pltpu.async_copy(
                rows.at[pl.ds(bi * 128, 128), pl.ds(0, cw)],
                acc.at[dbuf.at[sl, i]], ssems[bi], add=True)
    for i in range(16 - _NB, 16):
        sd[i % _NB].wait()


def _run_pass(tbl, src2, dst2, acc, sbuf, dbuf, rows, isems, gsems, ssems,
              s, n_ic, cw):
    base = s * (n_ic * 16)

    def fetch(row, sl):
        pltpu.async_copy(src2.at[pl.ds(row, 16), :], sbuf.at[sl], isems[0])
        pltpu.async_copy(dst2.at[pl.ds(row, 16), :], dbuf.at[sl], isems[1])

    def wait_fetch(row, sl):
        pltpu.make_async_copy(src2.at[pl.ds(row, 16), :], sbuf.at[sl],
                              isems[0]).wait()
        pltpu.make_async_copy(dst2.at[pl.ds(row, 16), :], dbuf.at[sl],
                              isems[1]).wait()

    fetch(base, 0)

    def pair_body(j, _):
        r0 = base + 2 * j * 16
        wait_fetch(r0, 0)
        fetch(r0 + 16, 1)
        _do_ic(tbl, acc, sbuf, dbuf, rows, gsems, ssems, 0, cw)
        wait_fetch(r0 + 16, 1)

        @pl.when(2 * j + 2 < n_ic)
        def _():
            fetch(r0 + 32, 0)

        _do_ic(tbl, acc, sbuf, dbuf, rows, gsems, ssems, 1, cw)
        return 0

    lax.fori_loop(0, n_ic // 2, pair_body, 0)


def _seg_phases(phases, acc, sbuf, dbuf, rows, zbuf, isems, gsems, ssems):
    c = lax.axis_index("c")
    s = lax.axis_index("s")
    zr = zbuf.shape[0]

    def zb(i, _):
        for k in range(zbuf.shape[1] // 16):
            zbuf[i, pl.ds(k * 16, 16)] = jnp.zeros((16,), jnp.float32)
        return 0
    lax.fori_loop(0, zr, zb, 0)

    for ytab, src2, dst2, out, n_dst, n_ic, n_pass, cw in phases:
        stripe = n_dst // 16
        for p in range(n_pass):
            for z in range(stripe // zr):
                pltpu.sync_copy(
                    zbuf.at[:, pl.ds(0, cw)],
                    acc.at[pl.ds(s * stripe + z * zr, zr), pl.ds(0, cw)])
            plsc.subcore_barrier()
            q = c * n_pass + p
            _run_pass(ytab.at[q], src2, dst2, acc, sbuf, dbuf, rows,
                      isems, gsems, ssems, s, n_ic, cw)
            plsc.subcore_barrier()
            pltpu.sync_copy(
                acc.at[pl.ds(s * stripe, stripe), pl.ds(0, cw)],
                out.at[pl.ds(s * stripe, stripe), pl.ds(cw * q, cw)])


def _ee_body(yee, se, sd, out, acc, sbuf, dbuf, rows, zbuf,
             isems, gsems, ssems):
    _seg_phases([(yee, se, sd, out, _NEP, 8, 2, 32)],
                acc, sbuf, dbuf, rows, zbuf, isems, gsems, ssems)


def _a_body(yea, yae, se_ea, sd_ea, se_ae, sd_ae, o_ea, o_ae,
            acc, sbuf, dbuf, rows, zbuf, isems, gsems, ssems):
    _seg_phases([(yea, se_ea, sd_ea, o_ea, _NAP, 4, 1, 64),
                 (yae, se_ae, sd_ae, o_ae, _NAP, 4, 1, 64)],
                acc, sbuf, dbuf, rows, zbuf, isems, gsems, ssems)


def _sc_scratch(n_dst, dc, zr):
    return [
        pltpu.VMEM_SHARED((n_dst + 8, dc), jnp.float32),
        pltpu.VMEM((2, 16, 128), jnp.int32),
        pltpu.VMEM((2, 16, 128), jnp.int32),
        pltpu.VMEM((_NB * 128, dc), jnp.float32),
        pltpu.VMEM((zr, dc), jnp.float32),
        [pltpu.SemaphoreType.DMA] * 2,
        [pltpu.SemaphoreType.DMA] * _NB,
        [pltpu.SemaphoreType.DMA] * _NB,
    ]


_SC_PARAMS = pltpu.CompilerParams(use_tc_tiling_on_sc=False,
                                  needs_layout_passes=False)
_SC_MESH = dict(core_axis_name="c", subcore_axis_name="s")


def _sc_ee(yee, se, sd):
    f = pl.kernel(_ee_body,
                  out_type=jax.ShapeDtypeStruct((_NEP, 128), jnp.float32),
                  mesh=plsc.VectorSubcoreMesh(**_SC_MESH),
                  scratch_types=_sc_scratch(_NEP, 32, 64),
                  compiler_params=_SC_PARAMS)
    return f(yee.reshape(4, _NEP, 32), se, sd)


def _sc_a(yea, yae, se_ea, sd_ea, se_ae, sd_ae):
    out_type = (
        jax.ShapeDtypeStruct((_NAP, 128), jnp.float32),
        jax.ShapeDtypeStruct((_NAP, 128), jnp.float32),
    )
    f = pl.kernel(_a_body, out_type=out_type,
                  mesh=plsc.VectorSubcoreMesh(**_SC_MESH),
                  scratch_types=_sc_scratch(_NAP, 64, 64),
                  compiler_params=_SC_PARAMS)
    return f(yea.reshape(2, _NEP, 64), yae.reshape(2, _NAP, 64),
             se_ea, sd_ea, se_ae, sd_ae)


_BN = 1024


def _split_writes(cat, out_refs, specs):
    col = 0
    for o_ref, (nc, cw) in zip(out_refs, specs):
        if nc == 1:
            o_ref[...] = cat[:, col:col + cw]
        else:
            rb = _BN * cw // 128
            g = 128 // cw
            o_ref[...] = jnp.stack([
                jnp.concatenate(
                    [cat[:, col + q * cw:col + (q + 1) * cw]
                     .reshape(rb, g, cw)[:, k, :] for k in range(g)],
                    axis=-1)
                for q in range(nc)])
        col += nc * cw


def _tab_shapes(n, specs):
    shp, bspec = [], []
    for nc, cw in specs:
        if nc == 1:
            shp.append(jax.ShapeDtypeStruct((n, cw), jnp.float32))
            bspec.append(pl.BlockSpec((_BN, cw), lambda i: (i, 0)))
        else:
            rb = _BN * cw // 128
            shp.append(jax.ShapeDtypeStruct((nc, n * cw // 128, 128),
                                            jnp.float32))
            bspec.append(pl.BlockSpec((nc, rb, 128), lambda i: (0, i, 0)))
    return shp, bspec


def _proj_cat_body(x_ref, w1_ref, b1_ref, wc_ref, *out_refs, specs):
    h = jnp.maximum(
        jnp.dot(x_ref[...], w1_ref[...],
                preferred_element_type=jnp.float32) + b1_ref[...], 0.0)
    cat = jnp.dot(h, wc_ref[...], preferred_element_type=jnp.float32)
    _split_writes(cat, out_refs, specs)


def _tc_proj_cat(x, w1, b1, wc, specs):
    n = x.shape[0]
    kcols = wc.shape[1]
    out_shape, out_specs = _tab_shapes(n, specs)
    return pl.pallas_call(
        functools.partial(_proj_cat_body, specs=specs),
        grid=(n // _BN,),
        in_specs=[
            pl.BlockSpec((_BN, _D), lambda i: (i, 0)),
            pl.BlockSpec((_D, _D), lambda i: (0, 0)),
            pl.BlockSpec((1, _D), lambda i: (0, 0)),
            pl.BlockSpec((_D, kcols), lambda i: (0, 0)),
        ],
        out_specs=out_specs,
        out_shape=out_shape,
    )(x, w1, b1, wc)


def _combine_cat_body(*refs, specs, has_ee, nb_a, final):
    it = iter(refs)
    m = jnp.zeros((_BN, _D), jnp.float32)
    if has_ee:
        m = m + next(it)[...] * next(it)[...]
    ma = next(it)[...] * next(it)[...]
    if has_ee:
        i = pl.program_id(0)
        ma = jnp.where(i < nb_a, ma, 0.0)
    m = m + ma
    r_ref = next(it)
    b_ref = next(it)
    wc_ref = next(it)
    h = jnp.maximum(m + r_ref[...] + b_ref[...], 0.0)
    cat = jnp.dot(h, wc_ref[...], preferred_element_type=jnp.float32)
    rest = list(it)
    if final:
        cat = cat + rest[0][...]
        rest = rest[1:]
    _split_writes(cat, rest, specs)


def _tc_combine_cat(s_ee, cnt_ee, s_a, cnt_a, r, b, wc, bo, specs):
    n = r.shape[0]
    has_ee = s_ee is not None
    nb_a = s_a.shape[0] // _BN
    kcols = wc.shape[1]
    in_specs = []
    args = []
    if has_ee:
        in_specs += [
            pl.BlockSpec((_BN, _D), lambda i: (i, 0)),
            pl.BlockSpec((_BN, 1), lambda i: (i, 0)),
        ]
        args += [s_ee, cnt_ee]
    cl = nb_a - 1
    in_specs += [
        pl.BlockSpec((_BN, _D), lambda i, cl=cl: (jnp.minimum(i, cl), 0)),
        pl.BlockSpec((_BN, 1), lambda i, cl=cl: (jnp.minimum(i, cl), 0)),
        pl.BlockSpec((_BN, _D), lambda i: (i, 0)),
        pl.BlockSpec((1, _D), lambda i: (0, 0)),
        pl.BlockSpec((_D, kcols), lambda i: (0, 0)),
    ]
    args += [s_a, cnt_a, r, b, wc]
    if bo is not None:
        in_specs.append(pl.BlockSpec((1, kcols), lambda i: (0, 0)))
        args.append(bo)
    out_shape, out_specs = _tab_shapes(n, specs)
    body = functools.partial(_combine_cat_body, specs=specs, has_ee=has_ee,
                             nb_a=nb_a, final=bo is not None)
    return pl.pallas_call(
        body, grid=(n // _BN,), in_specs=in_specs, out_specs=out_specs,
        out_shape=out_shape,
    )(*args)


_E_SPECS = [(4, 32), (2, 64), (1, 128)]
_A_SPECS = [(2, 64), (1, 128)]
_Z_SPECS = [(1, 128)]


def kernel(x_entity, x_attribute, params, edge_index_ee, edge_index_ae,
           edge_index_ea):
    We, be = params['lin']['entity']
    Wa, ba = params['lin']['attribute']
    Woe, boe = params['out']['entity']
    Woa, boa = params['out']['attribute']
    convs = params['convs']

    src_ee, dst_ee = _pad_edges(edge_index_ee, _EP_EE, _NEP)
    src_ae, dst_ae = _pad_edges(edge_index_ae, _EP_AE, _NAP)
    src_ea, dst_ea = _pad_edges(edge_index_ea, _EP_EA, _NAP)
    x_entity = jnp.pad(x_entity, ((0, _NEP - _NE), (0, 0)))
    x_attribute = jnp.pad(x_attribute, ((0, _NAP - _NA), (0, 0)))

    cnt_ee_p, cnt_ae_p, cnt_ea_p = _sc_counts(dst_ee, dst_ae, dst_ea)
    inv_ee = _tc_invcnt(cnt_ee_p, _NEP)
    inv_ae = _tc_invcnt(cnt_ae_p, _NAP)
    inv_ea = _tc_invcnt(cnt_ea_p, _NAP)

    def ewc(layer):
        Wl_ee, _, Wr_ee = layer['ee']
        Wl_ea, _, _ = layer['ea']
        _, _, Wr_ae = layer['ae']
        return jnp.concatenate([Wl_ee, Wl_ea, Wr_ee + Wr_ae], axis=1)

    def awc(layer):
        Wl_ae, _, _ = layer['ae']
        _, _, Wr_ea = layer['ea']
        return jnp.concatenate([Wl_ae, Wr_ea], axis=1)

    def ebias(layer):
        return (layer['ee'][1] + layer['ae'][1]).reshape(1, _D)

    def abias(layer):
        return layer['ea'][1].reshape(1, _D)

    yee, yea, r_e = _tc_proj_cat(x_entity, We, be.reshape(1, _D),
                                 ewc(convs[0]), _E_SPECS)
    yae, r_a = _tc_proj_cat(x_attribute, Wa, ba.reshape(1, _D),
                            awc(convs[0]), _A_SPECS)

    for li in range(2):
        s_ee = _sc_ee(yee, src_ee, dst_ee)
        s_ea, s_ae = _sc_a(yea, yae, src_ea, dst_ea, src_ae, dst_ae)
        if li == 0:
            yee, yea, r_e = _tc_combine_cat(
                s_ee, inv_ee, s_ae, inv_ae, r_e, ebias(convs[0]),
                ewc(convs[1]), None, _E_SPECS)
            yae, r_a = _tc_combine_cat(
                None, None, s_ea, inv_ea, r_a, abias(convs[0]),
                awc(convs[1]), None, _A_SPECS)
        else:
            (z_e,) = _tc_combine_cat(
                s_ee, inv_ee, s_ae, inv_ae, r_e, ebias(convs[1]),
                Woe, boe.reshape(1, _D), _Z_SPECS)
            (z_a,) = _tc_combine_cat(
                None, None, s_ea, inv_ea, r_a, abias(convs[1]),
                Woa, boa.reshape(1, _D), _Z_SPECS)

    return (z_e[:_NE], z_a[:_NA])


# speedup vs baseline: 1.4148x; 1.0065x over previous
import functools

import jax
import jax.numpy as jnp
from jax import lax
from jax.experimental import pallas as pl
from jax.experimental.pallas import tpu as pltpu
from jax.experimental.pallas import tpu_sc as plsc

_NE = 50000
_NA = 10000
_NEP = 51200
_NAP = 10240
_D = 128

_EP_EE = 262144
_EP_AE = 131072
_EP_EA = 131072

_LC_E = 51456
_LC_A = 10368

_NB = 4
_LAG = 3
_NBA = 8
_LAGA = 4


def _pad_edges(ei, e_pad, dummy_dst):
    src = jnp.pad(ei[0], (0, e_pad - ei.shape[1]))
    dst = jnp.pad(ei[1], (0, e_pad - ei.shape[1]), constant_values=dummy_dst)
    return src.reshape(e_pad // 128, 128), dst.reshape(e_pad // 128, 128)


def _counts_body(dee_hbm, dae_hbm, dea_hbm, oee, oae, oea,
                 cee, cae, cea, dchunk):
    c = lax.axis_index("c")
    s = lax.axis_index("s")
    tid = c * 16 + s

    ones = jnp.ones((16,), jnp.float32)
    zeros = jnp.zeros((16,), jnp.float32)
    m127 = jnp.full((16,), 127, jnp.int32)

    def _z(ref):
        def body(i, _):
            for k in range(8):
                ref[i, pl.ds(k * 16, 16)] = zeros
            return 0
        lax.fori_loop(0, ref.shape[0], body, 0)
    _z(cee)
    _z(cae)
    _z(cea)

    def _hist(dst_hbm, cnt_ref, rows_per_tile):
        def ic_body(ic, _):
            r0 = tid * rows_per_tile + ic * 16
            pltpu.sync_copy(dst_hbm.at[pl.ds(r0, 16), :], dchunk)

            def row_body(j, _):
                for k in range(8):
                    d16 = dchunk[j, pl.ds(k * 16, 16)]
                    plsc.addupdate_scatter(
                        cnt_ref, [lax.shift_right_logical(d16, 7),
                                  lax.bitwise_and(d16, m127)], ones)
                return 0
            lax.fori_loop(0, 16, row_body, 0)
            return 0
        lax.fori_loop(0, rows_per_tile // 16, ic_body, 0)

    _hist(dee_hbm, cee, _EP_EE // 128 // 32)
    _hist(dae_hbm, cae, _EP_AE // 128 // 32)
    _hist(dea_hbm, cea, _EP_EA // 128 // 32)

    pltpu.sync_copy(cee, oee.at[tid])
    pltpu.sync_copy(cae, oae.at[tid])
    pltpu.sync_copy(cea, oea.at[tid])


def _sc_counts(dst2_ee, dst2_ae, dst2_ea):
    mesh = plsc.VectorSubcoreMesh(core_axis_name="c", subcore_axis_name="s")
    out_type = (
        jax.ShapeDtypeStruct((32, _LC_E // 128, 128), jnp.float32),
        jax.ShapeDtypeStruct((32, _LC_A // 128, 128), jnp.float32),
        jax.ShapeDtypeStruct((32, _LC_A // 128, 128), jnp.float32),
    )
    scratch = [
        pltpu.VMEM((_LC_E // 128, 128), jnp.float32),
        pltpu.VMEM((_LC_A // 128, 128), jnp.float32),
        pltpu.VMEM((_LC_A // 128, 128), jnp.float32),
        pltpu.VMEM((16, 128), jnp.int32),
    ]
    f = pl.kernel(_counts_body, out_type=out_type, mesh=mesh,
                  scratch_types=scratch,
                  compiler_params=pltpu.CompilerParams(
                      use_tc_tiling_on_sc=False, needs_layout_passes=False))
    return f(dst2_ee, dst2_ae, dst2_ea)


def _inv_body(p_ref, o_ref):
    s = jnp.sum(p_ref[...], axis=0)
    o_ref[...] = 1.0 / jnp.maximum(s, 1.0)


def _tc_invcnt(p, n):
    rows = p.shape[1]
    out = pl.pallas_call(
        _inv_body,
        out_shape=jax.ShapeDtypeStruct((rows, 128), jnp.float32),
    )(p)
    return out.reshape(rows * 128, 1)[:n]


def _do_ic(tbl, acc, sbuf, dbuf, rows, gsems, ssems, sl, cw, nb, lag):
    gd = [None] * nb
    sd = [None] * nb
    for t in range(16 + lag):
        if t < 16:
            b = t % nb
            if t >= nb:
                sd[b].wait()
            gd[b] = pltpu.async_copy(
                tbl.at[sbuf.at[sl, t]],
                rows.at[pl.ds(b * 128, 128), pl.ds(0, cw)], gsems[b])
        if t >= lag:
            i = t - lag
            bi = i % nb
            gd[bi].wait()
            sd[bi] = pltpu.async_copy(
                rows.at[pl.ds(bi * 128, 128), pl.ds(0, cw)],
                acc.at[dbuf.at[sl, i]], ssems[bi], add=True)
    for i in range(16 - nb, 16):
        sd[i % nb].wait()


def _run_pass(tbl, src2, dst2, acc, sbuf, dbuf, rows, isems, gsems, ssems,
              s, n_ic, cw, nb, lag):
    base = s * (n_ic * 16)

    def fetch(row, sl):
        pltpu.async_copy(src2.at[pl.ds(row, 16), :], sbuf.at[sl], isems[0])
        pltpu.async_copy(dst2.at[pl.ds(row, 16), :], dbuf.at[sl], isems[1])

    def wait_fetch(row, sl):
        pltpu.make_async_copy(src2.at[pl.ds(row, 16), :], sbuf.at[sl],
                              isems[0]).wait()
        pltpu.make_async_copy(dst2.at[pl.ds(row, 16), :], dbuf.at[sl],
                              isems[1]).wait()

    fetch(base, 0)

    def pair_body(j, _):
        r0 = base + 2 * j * 16
        wait_fetch(r0, 0)
        fetch(r0 + 16, 1)
        _do_ic(tbl, acc, sbuf, dbuf, rows, gsems, ssems, 0, cw, nb, lag)
        wait_fetch(r0 + 16, 1)

        @pl.when(2 * j + 2 < n_ic)
        def _():
            fetch(r0 + 32, 0)

        _do_ic(tbl, acc, sbuf, dbuf, rows, gsems, ssems, 1, cw, nb, lag)
        return 0

    lax.fori_loop(0, n_ic // 2, pair_body, 0)


def _seg_phases(phases, acc, sbuf, dbuf, rows, zbuf, isems, gsems, ssems,
                nb, lag):
    c = lax.axis_index("c")
    s = lax.axis_index("s")
    zr = zbuf.shape[0]

    def zb(i, _):
        for k in range(zbuf.shape[1] // 16):
            zbuf[i, pl.ds(k * 16, 16)] = jnp.zeros((16,), jnp.float32)
        return 0
    lax.fori_loop(0, zr, zb, 0)

    for ytab, src2, dst2, out, n_dst, n_ic, n_pass, cw in phases:
        stripe = n_dst // 16
        for p in range(n_pass):
            for z in range(stripe // zr):
                pltpu.sync_copy(
                    zbuf.at[:, pl.ds(0, cw)],
                    acc.at[pl.ds(s * stripe + z * zr, zr), pl.ds(0, cw)])
            plsc.subcore_barrier()
            q = c * n_pass + p
            _run_pass(ytab.at[q], src2, dst2, acc, sbuf, dbuf, rows,
                      isems, gsems, ssems, s, n_ic, cw, nb, lag)
            plsc.subcore_barrier()
            pltpu.sync_copy(
                acc.at[pl.ds(s * stripe, stripe), pl.ds(0, cw)],
                out.at[pl.ds(s * stripe, stripe), pl.ds(cw * q, cw)])


def _ee_body(yee, se, sd, out, acc, sbuf, dbuf, rows, zbuf,
             isems, gsems, ssems):
    _seg_phases([(yee, se, sd, out, _NEP, 8, 2, 32)],
                acc, sbuf, dbuf, rows, zbuf, isems, gsems, ssems, _NB, _LAG)


def _a_body(yea, yae, se_ea, sd_ea, se_ae, sd_ae, o_ea, o_ae,
            acc, sbuf, dbuf, rows, zbuf, isems, gsems, ssems):
    _seg_phases([(yea, se_ea, sd_ea, o_ea, _NAP, 4, 1, 64),
                 (yae, se_ae, sd_ae, o_ae, _NAP, 4, 1, 64)],
                acc, sbuf, dbuf, rows, zbuf, isems, gsems, ssems, _NBA, _LAGA)


def _sc_scratch(n_dst, dc, zr, nb):
    return [
        pltpu.VMEM_SHARED((n_dst + 8, dc), jnp.float32),
        pltpu.VMEM((2, 16, 128), jnp.int32),
        pltpu.VMEM((2, 16, 128), jnp.int32),
        pltpu.VMEM((nb * 128, dc), jnp.float32),
        pltpu.VMEM((zr, dc), jnp.float32),
        [pltpu.SemaphoreType.DMA] * 2,
        [pltpu.SemaphoreType.DMA] * nb,
        [pltpu.SemaphoreType.DMA] * nb,
    ]


_SC_PARAMS = pltpu.CompilerParams(use_tc_tiling_on_sc=False,
                                  needs_layout_passes=False)
_SC_MESH = dict(core_axis_name="c", subcore_axis_name="s")


def _sc_ee(yee, se, sd):
    f = pl.kernel(_ee_body,
                  out_type=jax.ShapeDtypeStruct((_NEP, 128), jnp.float32),
                  mesh=plsc.VectorSubcoreMesh(**_SC_MESH),
                  scratch_types=_sc_scratch(_NEP, 32, 64, _NB),
                  compiler_params=_SC_PARAMS)
    return f(yee.reshape(4, _NEP, 32), se, sd)


def _sc_a(yea, yae, se_ea, sd_ea, se_ae, sd_ae):
    out_type = (
        jax.ShapeDtypeStruct((_NAP, 128), jnp.float32),
        jax.ShapeDtypeStruct((_NAP, 128), jnp.float32),
    )
    f = pl.kernel(_a_body, out_type=out_type,
                  mesh=plsc.VectorSubcoreMesh(**_SC_MESH),
                  scratch_types=_sc_scratch(_NAP, 64, 64, _NBA),
                  compiler_params=_SC_PARAMS)
    return f(yea.reshape(2, _NEP, 64), yae.reshape(2, _NAP, 64),
             se_ea, sd_ea, se_ae, sd_ae)


_BN = 1024


def _split_writes(cat, out_refs, specs):
    col = 0
    for o_ref, (nc, cw) in zip(out_refs, specs):
        if nc == 1:
            o_ref[...] = cat[:, col:col + cw]
        else:
            rb = _BN * cw // 128
            g = 128 // cw
            o_ref[...] = jnp.stack([
                jnp.concatenate(
                    [cat[:, col + q * cw:col + (q + 1) * cw]
                     .reshape(rb, g, cw)[:, k, :] for k in range(g)],
                    axis=-1)
                for q in range(nc)])
        col += nc * cw


def _tab_shapes(n, specs):
    shp, bspec = [], []
    for nc, cw in specs:
        if nc == 1:
            shp.append(jax.ShapeDtypeStruct((n, cw), jnp.float32))
            bspec.append(pl.BlockSpec((_BN, cw), lambda i: (i, 0)))
        else:
            rb = _BN * cw // 128
            shp.append(jax.ShapeDtypeStruct((nc, n * cw // 128, 128),
                                            jnp.float32))
            bspec.append(pl.BlockSpec((nc, rb, 128), lambda i: (0, i, 0)))
    return shp, bspec


def _proj_cat_body(x_ref, w1_ref, b1_ref, wc_ref, *out_refs, specs):
    h = jnp.maximum(
        jnp.dot(x_ref[...], w1_ref[...],
                preferred_element_type=jnp.float32) + b1_ref[...], 0.0)
    cat = jnp.dot(h, wc_ref[...], preferred_element_type=jnp.float32)
    _split_writes(cat, out_refs, specs)


def _tc_proj_cat(x, w1, b1, wc, specs):
    n = x.shape[0]
    kcols = wc.shape[1]
    out_shape, out_specs = _tab_shapes(n, specs)
    return pl.pallas_call(
        functools.partial(_proj_cat_body, specs=specs),
        grid=(n // _BN,),
        in_specs=[
            pl.BlockSpec((_BN, _D), lambda i: (i, 0)),
            pl.BlockSpec((_D, _D), lambda i: (0, 0)),
            pl.BlockSpec((1, _D), lambda i: (0, 0)),
            pl.BlockSpec((_D, kcols), lambda i: (0, 0)),
        ],
        out_specs=out_specs,
        out_shape=out_shape,
    )(x, w1, b1, wc)


def _combine_cat_body(*refs, specs, has_ee, nb_a, final):
    it = iter(refs)
    m = jnp.zeros((_BN, _D), jnp.float32)
    if has_ee:
        m = m + next(it)[...] * next(it)[...]
    ma = next(it)[...] * next(it)[...]
    if has_ee:
        i = pl.program_id(0)
        ma = jnp.where(i < nb_a, ma, 0.0)
    m = m + ma
    r_ref = next(it)
    b_ref = next(it)
    wc_ref = next(it)
    h = jnp.maximum(m + r_ref[...] + b_ref[...], 0.0)
    cat = jnp.dot(h, wc_ref[...], preferred_element_type=jnp.float32)
    rest = list(it)
    if final:
        cat = cat + rest[0][...]
        rest = rest[1:]
    _split_writes(cat, rest, specs)


def _tc_combine_cat(s_ee, cnt_ee, s_a, cnt_a, r, b, wc, bo, specs):
    n = r.shape[0]
    has_ee = s_ee is not None
    nb_a = s_a.shape[0] // _BN
    kcols = wc.shape[1]
    in_specs = []
    args = []
    if has_ee:
        in_specs += [
            pl.BlockSpec((_BN, _D), lambda i: (i, 0)),
            pl.BlockSpec((_BN, 1), lambda i: (i, 0)),
        ]
        args += [s_ee, cnt_ee]
    cl = nb_a - 1
    in_specs += [
        pl.BlockSpec((_BN, _D), lambda i, cl=cl: (jnp.minimum(i, cl), 0)),
        pl.BlockSpec((_BN, 1), lambda i, cl=cl: (jnp.minimum(i, cl), 0)),
        pl.BlockSpec((_BN, _D), lambda i: (i, 0)),
        pl.BlockSpec((1, _D), lambda i: (0, 0)),
        pl.BlockSpec((_D, kcols), lambda i: (0, 0)),
    ]
    args += [s_a, cnt_a, r, b, wc]
    if bo is not None:
        in_specs.append(pl.BlockSpec((1, kcols), lambda i: (0, 0)))
        args.append(bo)
    out_shape, out_specs = _tab_shapes(n, specs)
    body = functools.partial(_combine_cat_body, specs=specs, has_ee=has_ee,
                             nb_a=nb_a, final=bo is not None)
    return pl.pallas_call(
        body, grid=(n // _BN,), in_specs=in_specs, out_specs=out_specs,
        out_shape=out_shape,
    )(*args)


_E_SPECS = [(4, 32), (2, 64), (1, 128)]
_A_SPECS = [(2, 64), (1, 128)]
_Z_SPECS = [(1, 128)]


def kernel(x_entity, x_attribute, params, edge_index_ee, edge_index_ae,
           edge_index_ea):
    We, be = params['lin']['entity']
    Wa, ba = params['lin']['attribute']
    Woe, boe = params['out']['entity']
    Woa, boa = params['out']['attribute']
    convs = params['convs']

    src_ee, dst_ee = _pad_edges(edge_index_ee, _EP_EE, _NEP)
    src_ae, dst_ae = _pad_edges(edge_index_ae, _EP_AE, _NAP)
    src_ea, dst_ea = _pad_edges(edge_index_ea, _EP_EA, _NAP)
    x_entity = jnp.pad(x_entity, ((0, _NEP - _NE), (0, 0)))
    x_attribute = jnp.pad(x_attribute, ((0, _NAP - _NA), (0, 0)))

    cnt_ee_p, cnt_ae_p, cnt_ea_p = _sc_counts(dst_ee, dst_ae, dst_ea)
    inv_ee = _tc_invcnt(cnt_ee_p, _NEP)
    inv_ae = _tc_invcnt(cnt_ae_p, _NAP)
    inv_ea = _tc_invcnt(cnt_ea_p, _NAP)

    def ewc(layer):
        Wl_ee, _, Wr_ee = layer['ee']
        Wl_ea, _, _ = layer['ea']
        _, _, Wr_ae = layer['ae']
        return jnp.concatenate([Wl_ee, Wl_ea, Wr_ee + Wr_ae], axis=1)

    def awc(layer):
        Wl_ae, _, _ = layer['ae']
        _, _, Wr_ea = layer['ea']
        return jnp.concatenate([Wl_ae, Wr_ea], axis=1)

    def ebias(layer):
        return (layer['ee'][1] + layer['ae'][1]).reshape(1, _D)

    def abias(layer):
        return layer['ea'][1].reshape(1, _D)

    yee, yea, r_e = _tc_proj_cat(x_entity, We, be.reshape(1, _D),
                                 ewc(convs[0]), _E_SPECS)
    yae, r_a = _tc_proj_cat(x_attribute, Wa, ba.reshape(1, _D),
                            awc(convs[0]), _A_SPECS)

    for li in range(2):
        s_ee = _sc_ee(yee, src_ee, dst_ee)
        s_ea, s_ae = _sc_a(yea, yae, src_ea, dst_ea, src_ae, dst_ae)
        if li == 0:
            yee, yea, r_e = _tc_combine_cat(
                s_ee, inv_ee, s_ae, inv_ae, r_e, ebias(convs[0]),
                ewc(convs[1]), None, _E_SPECS)
            yae, r_a = _tc_combine_cat(
                None, None, s_ea, inv_ea, r_a, abias(convs[0]),
                awc(convs[1]), None, _A_SPECS)
        else:
            (z_e,) = _tc_combine_cat(
                s_ee, inv_ee, s_ae, inv_ae, r_e, ebias(convs[1]),
                Woe, boe.reshape(1, _D), _Z_SPECS)
            (z_a,) = _tc_combine_cat(
                None, None, s_ea, inv_ea, r_a, abias(convs[1]),
                Woa, boa.reshape(1, _D), _Z_SPECS)

    return (z_e[:_NE], z_a[:_NA])


# speedup vs baseline: 1.4465x; 1.0224x over previous
import functools

import jax
import jax.numpy as jnp
from jax import lax
from jax.experimental import pallas as pl
from jax.experimental.pallas import tpu as pltpu
from jax.experimental.pallas import tpu_sc as plsc

_NE = 50000
_NA = 10000
_NEP = 51200
_NAP = 10240
_D = 128

_EP_EE = 262144
_EP_AE = 131072
_EP_EA = 131072

_LC_E = 51456
_LC_A = 10368

_NB = 4
_LAG = 3
_NBA = 8
_LAGA = 4


def _pad_edges(ei, e_pad, dummy_dst):
    src = jnp.pad(ei[0], (0, e_pad - ei.shape[1]))
    dst = jnp.pad(ei[1], (0, e_pad - ei.shape[1]), constant_values=dummy_dst)
    return src.reshape(e_pad // 128, 128), dst.reshape(e_pad // 128, 128)


def _counts_body(dee_hbm, dae_hbm, dea_hbm, oee, oae, oea,
                 cee, cae, cea, dchunk):
    c = lax.axis_index("c")
    s = lax.axis_index("s")
    tid = c * 16 + s

    ones = jnp.ones((16,), jnp.float32)
    zeros = jnp.zeros((16,), jnp.float32)
    m127 = jnp.full((16,), 127, jnp.int32)

    def _z(ref):
        def body(i, _):
            for k in range(8):
                ref[i, pl.ds(k * 16, 16)] = zeros
            return 0
        lax.fori_loop(0, ref.shape[0], body, 0)
    _z(cee)
    _z(cae)
    _z(cea)

    def _hist(dst_hbm, cnt_ref, rows_per_tile):
        def ic_body(ic, _):
            r0 = tid * rows_per_tile + ic * 16
            pltpu.sync_copy(dst_hbm.at[pl.ds(r0, 16), :], dchunk)

            def row_body(j, _):
                for k in range(8):
                    d16 = dchunk[j, pl.ds(k * 16, 16)]
                    plsc.addupdate_scatter(
                        cnt_ref, [lax.shift_right_logical(d16, 7),
                                  lax.bitwise_and(d16, m127)], ones)
                return 0
            lax.fori_loop(0, 16, row_body, 0)
            return 0
        lax.fori_loop(0, rows_per_tile // 16, ic_body, 0)

    _hist(dee_hbm, cee, _EP_EE // 128 // 32)
    _hist(dae_hbm, cae, _EP_AE // 128 // 32)
    _hist(dea_hbm, cea, _EP_EA // 128 // 32)

    pltpu.sync_copy(cee, oee.at[tid])
    pltpu.sync_copy(cae, oae.at[tid])
    pltpu.sync_copy(cea, oea.at[tid])


def _sc_counts(dst2_ee, dst2_ae, dst2_ea):
    mesh = plsc.VectorSubcoreMesh(core_axis_name="c", subcore_axis_name="s")
    out_type = (
        jax.ShapeDtypeStruct((32, _LC_E // 128, 128), jnp.float32),
        jax.ShapeDtypeStruct((32, _LC_A // 128, 128), jnp.float32),
        jax.ShapeDtypeStruct((32, _LC_A // 128, 128), jnp.float32),
    )
    scratch = [
        pltpu.VMEM((_LC_E // 128, 128), jnp.float32),
        pltpu.VMEM((_LC_A // 128, 128), jnp.float32),
        pltpu.VMEM((_LC_A // 128, 128), jnp.float32),
        pltpu.VMEM((16, 128), jnp.int32),
    ]
    f = pl.kernel(_counts_body, out_type=out_type, mesh=mesh,
                  scratch_types=scratch,
                  compiler_params=pltpu.CompilerParams(
                      use_tc_tiling_on_sc=False, needs_layout_passes=False))
    return f(dst2_ee, dst2_ae, dst2_ea)


def _inv_body(p_ref, o_ref):
    s = jnp.sum(p_ref[...], axis=0)
    o_ref[...] = 1.0 / jnp.maximum(s, 1.0)


def _tc_invcnt(p, n):
    rows = p.shape[1]
    out = pl.pallas_call(
        _inv_body,
        out_shape=jax.ShapeDtypeStruct((rows, 128), jnp.float32),
    )(p)
    return out.reshape(rows * 128, 1)[:n]


def _do_ic(tbl, acc, sbuf, dbuf, rows, gsems, ssems, sl, cw, nb, lag):
    gd = [None] * nb
    sd = [None] * nb
    for t in range(16 + lag):
        if t < 16:
            b = t % nb
            if t >= nb:
                sd[b].wait()
            gd[b] = pltpu.async_copy(
                tbl.at[sbuf.at[sl, t]],
                rows.at[pl.ds(b * 128, 128), pl.ds(0, cw)], gsems[b])
        if t >= lag:
            i = t - lag
            bi = i % nb
            gd[bi].wait()
            sd[bi] = pltpu.async_copy(
                rows.at[pl.ds(bi * 128, 128), pl.ds(0, cw)],
                acc.at[dbuf.at[sl, i]], ssems[bi], add=True)
    for i in range(16 - nb, 16):
        sd[i % nb].wait()


def _run_pass(tbl, src2, dst2, acc, sbuf, dbuf, rows, isems, gsems, ssems,
              s, n_ic, cw, nb, lag):
    base = s * (n_ic * 16)

    def fetch(row, sl):
        pltpu.async_copy(src2.at[pl.ds(row, 16), :], sbuf.at[sl], isems[0])
        pltpu.async_copy(dst2.at[pl.ds(row, 16), :], dbuf.at[sl], isems[1])

    def wait_fetch(row, sl):
        pltpu.make_async_copy(src2.at[pl.ds(row, 16), :], sbuf.at[sl],
                              isems[0]).wait()
        pltpu.make_async_copy(dst2.at[pl.ds(row, 16), :], dbuf.at[sl],
                              isems[1]).wait()

    fetch(base, 0)

    def pair_body(j, _):
        r0 = base + 2 * j * 16
        wait_fetch(r0, 0)
        fetch(r0 + 16, 1)
        _do_ic(tbl, acc, sbuf, dbuf, rows, gsems, ssems, 0, cw, nb, lag)
        wait_fetch(r0 + 16, 1)

        @pl.when(2 * j + 2 < n_ic)
        def _():
            fetch(r0 + 32, 0)

        _do_ic(tbl, acc, sbuf, dbuf, rows, gsems, ssems, 1, cw, nb, lag)
        return 0

    lax.fori_loop(0, n_ic // 2, pair_body, 0)


def _seg_phases(phases, acc, sbuf, dbuf, rows, zbuf, isems, gsems, ssems,
                nb, lag):
    c = lax.axis_index("c")
    s = lax.axis_index("s")
    zr = zbuf.shape[0]

    def zb(i, _):
        for k in range(zbuf.shape[1] // 16):
            zbuf[i, pl.ds(k * 16, 16)] = jnp.zeros((16,), jnp.float32)
        return 0
    lax.fori_loop(0, zr, zb, 0)

    for ytab, src2, dst2, out, n_dst, n_ic, n_pass, cw in phases:
        stripe = n_dst // 16
        for p in range(n_pass):
            for z in range(stripe // zr):
                pltpu.sync_copy(
                    zbuf.at[:, pl.ds(0, cw)],
                    acc.at[pl.ds(s * stripe + z * zr, zr), pl.ds(0, cw)])
            plsc.subcore_barrier()
            q = c * n_pass + p
            _run_pass(ytab.at[q], src2, dst2, acc, sbuf, dbuf, rows,
                      isems, gsems, ssems, s, n_ic, cw, nb, lag)
            plsc.subcore_barrier()
            pltpu.sync_copy(
                acc.at[pl.ds(s * stripe, stripe), pl.ds(0, cw)],
                out.at[pl.ds(s * stripe, stripe), pl.ds(cw * q, cw)])


def _ee_body(yee, se, sd, out, acc, sbuf, dbuf, rows, zbuf,
             isems, gsems, ssems):
    _seg_phases([(yee, se, sd, out, _NEP, 8, 2, 32)],
                acc, sbuf, dbuf, rows, zbuf, isems, gsems, ssems, _NB, _LAG)


def _a_body(yea, yae, se_ea, sd_ea, se_ae, sd_ae, o_ea, o_ae,
            acc, sbuf, dbuf, rows, zbuf, isems, gsems, ssems):
    _seg_phases([(yea, se_ea, sd_ea, o_ea, _NAP, 4, 1, 64),
                 (yae, se_ae, sd_ae, o_ae, _NAP, 4, 1, 64)],
                acc, sbuf, dbuf, rows, zbuf, isems, gsems, ssems, _NBA, _LAGA)


def _sc_scratch(n_dst, dc, zr, nb):
    return [
        pltpu.VMEM_SHARED((n_dst + 8, dc), jnp.float32),
        pltpu.VMEM((2, 16, 128), jnp.int32),
        pltpu.VMEM((2, 16, 128), jnp.int32),
        pltpu.VMEM((nb * 128, dc), jnp.float32),
        pltpu.VMEM((zr, dc), jnp.float32),
        [pltpu.SemaphoreType.DMA] * 2,
        [pltpu.SemaphoreType.DMA] * nb,
        [pltpu.SemaphoreType.DMA] * nb,
    ]


_SC_PARAMS = pltpu.CompilerParams(use_tc_tiling_on_sc=False,
                                  needs_layout_passes=False)
_SC_MESH = dict(core_axis_name="c", subcore_axis_name="s")


def _sc_ee(yee, se, sd):
    f = pl.kernel(_ee_body,
                  out_type=jax.ShapeDtypeStruct((_NEP, 128), jnp.float32),
                  mesh=plsc.VectorSubcoreMesh(**_SC_MESH),
                  scratch_types=_sc_scratch(_NEP, 32, 64, _NB),
                  compiler_params=_SC_PARAMS)
    return f(yee.reshape(4, _NEP, 32), se, sd)


def _sc_a(yea, yae, se_ea, sd_ea, se_ae, sd_ae):
    out_type = (
        jax.ShapeDtypeStruct((_NAP, 128), jnp.float32),
        jax.ShapeDtypeStruct((_NAP, 128), jnp.float32),
    )
    f = pl.kernel(_a_body, out_type=out_type,
                  mesh=plsc.VectorSubcoreMesh(**_SC_MESH),
                  scratch_types=_sc_scratch(_NAP, 64, 64, _NBA),
                  compiler_params=_SC_PARAMS)
    return f(yea.reshape(2, _NEP, 64), yae.reshape(2, _NAP, 64),
             se_ea, sd_ea, se_ae, sd_ae)


_BN = 1024


def _split_writes(cat, out_refs, specs):
    col = 0
    for o_ref, (nc, cw) in zip(out_refs, specs):
        if nc == 1:
            o_ref[...] = cat[:, col:col + cw]
        else:
            rb = _BN * cw // 128
            g = 128 // cw
            o_ref[...] = jnp.stack([
                jnp.concatenate(
                    [cat[:, col + q * cw:col + (q + 1) * cw]
                     .reshape(rb, g, cw)[:, k, :] for k in range(g)],
                    axis=-1)
                for q in range(nc)])
        col += nc * cw


def _tab_shapes(n, specs):
    shp, bspec = [], []
    for nc, cw in specs:
        if nc == 1:
            shp.append(jax.ShapeDtypeStruct((n, cw), jnp.float32))
            bspec.append(pl.BlockSpec((_BN, cw), lambda i: (i, 0)))
        else:
            rb = _BN * cw // 128
            shp.append(jax.ShapeDtypeStruct((nc, n * cw // 128, 128),
                                            jnp.float32))
            bspec.append(pl.BlockSpec((nc, rb, 128), lambda i: (0, i, 0)))
    return shp, bspec


def _proj_cat_body(x_ref, w1_ref, b1_ref, wc_ref, *out_refs, specs):
    h = jnp.maximum(
        jnp.dot(x_ref[...], w1_ref[...],
                preferred_element_type=jnp.float32) + b1_ref[...], 0.0)
    cat = jnp.dot(h, wc_ref[...], preferred_element_type=jnp.float32)
    _split_writes(cat, out_refs, specs)


def _tc_proj_cat(x, w1, b1, wc, specs):
    n = x.shape[0]
    kcols = wc.shape[1]
    out_shape, out_specs = _tab_shapes(n, specs)
    return pl.pallas_call(
        functools.partial(_proj_cat_body, specs=specs),
        grid=(n // _BN,),
        in_specs=[
            pl.BlockSpec((_BN, _D), lambda i: (i, 0)),
            pl.BlockSpec((_D, _D), lambda i: (0, 0)),
            pl.BlockSpec((1, _D), lambda i: (0, 0)),
            pl.BlockSpec((_D, kcols), lambda i: (0, 0)),
        ],
        out_specs=out_specs,
        out_shape=out_shape,
    )(x, w1, b1, wc)


def _combine_cat_body(*refs, specs, has_ee, has_a, nb_a, final, need_mask):
    it = iter(refs)
    m = jnp.zeros((_BN, _D), jnp.float32)
    if has_ee:
        m = m + next(it)[...] * next(it)[...]
    if has_a:
        ma = next(it)[...] * next(it)[...]
        if need_mask:
            i = pl.program_id(0)
            ma = jnp.where(i < nb_a, ma, 0.0)
        m = m + ma
    r_ref = next(it)
    b_ref = next(it)
    wc_ref = next(it)
    h = jnp.maximum(m + r_ref[...] + b_ref[...], 0.0)
    cat = jnp.dot(h, wc_ref[...], preferred_element_type=jnp.float32)
    rest = list(it)
    if final:
        cat = cat + rest[0][...]
        rest = rest[1:]
    _split_writes(cat, rest, specs)


def _tc_combine_cat(s_ee, cnt_ee, s_a, cnt_a, r, b, wc, bo, specs,
                    sb=0, nblocks=None):
    n = r.shape[0]
    if nblocks is None:
        nblocks = n // _BN - sb
    has_ee = s_ee is not None
    has_a = s_a is not None
    nb_a = s_a.shape[0] // _BN if has_a else 0
    need_mask = has_a and (sb + nblocks > nb_a)
    kcols = wc.shape[1]
    in_specs = []
    args = []
    if has_ee:
        in_specs += [
            pl.BlockSpec((_BN, _D), lambda i: (i + sb, 0)),
            pl.BlockSpec((_BN, 1), lambda i: (i + sb, 0)),
        ]
        args += [s_ee, cnt_ee]
    if has_a:
        cl = nb_a - 1
        in_specs += [
            pl.BlockSpec((_BN, _D),
                         lambda i, cl=cl: (jnp.minimum(i + sb, cl), 0)),
            pl.BlockSpec((_BN, 1),
                         lambda i, cl=cl: (jnp.minimum(i + sb, cl), 0)),
        ]
        args += [s_a, cnt_a]
    in_specs += [
        pl.BlockSpec((_BN, _D), lambda i: (i + sb, 0)),
        pl.BlockSpec((1, _D), lambda i: (0, 0)),
        pl.BlockSpec((_D, kcols), lambda i: (0, 0)),
    ]
    args += [r, b, wc]
    if bo is not None:
        in_specs.append(pl.BlockSpec((1, kcols), lambda i: (0, 0)))
        args.append(bo)
    out_shape, out_specs = _tab_shapes(nblocks * _BN, specs)
    body = functools.partial(_combine_cat_body, specs=specs, has_ee=has_ee,
                             has_a=has_a, nb_a=nb_a, final=bo is not None,
                             need_mask=need_mask)
    return pl.pallas_call(
        body, grid=(nblocks,), in_specs=in_specs, out_specs=out_specs,
        out_shape=out_shape,
    )(*args)


_E_SPECS = [(4, 32), (2, 64), (1, 128)]
_A_SPECS = [(2, 64), (1, 128)]
_Z_SPECS = [(1, 128)]


def kernel(x_entity, x_attribute, params, edge_index_ee, edge_index_ae,
           edge_index_ea):
    We, be = params['lin']['entity']
    Wa, ba = params['lin']['attribute']
    Woe, boe = params['out']['entity']
    Woa, boa = params['out']['attribute']
    convs = params['convs']

    src_ee, dst_ee = _pad_edges(edge_index_ee, _EP_EE, _NEP)
    src_ae, dst_ae = _pad_edges(edge_index_ae, _EP_AE, _NAP)
    src_ea, dst_ea = _pad_edges(edge_index_ea, _EP_EA, _NAP)
    x_entity = jnp.pad(x_entity, ((0, _NEP - _NE), (0, 0)))
    x_attribute = jnp.pad(x_attribute, ((0, _NAP - _NA), (0, 0)))

    cnt_ee_p, cnt_ae_p, cnt_ea_p = _sc_counts(dst_ee, dst_ae, dst_ea)
    inv_ee = _tc_invcnt(cnt_ee_p, _NEP)
    inv_ae = _tc_invcnt(cnt_ae_p, _NAP)
    inv_ea = _tc_invcnt(cnt_ea_p, _NAP)

    def ewc(layer):
        Wl_ee, _, Wr_ee = layer['ee']
        Wl_ea, _, _ = layer['ea']
        _, _, Wr_ae = layer['ae']
        return jnp.concatenate([Wl_ee, Wl_ea, Wr_ee + Wr_ae], axis=1)

    def ewc2(layer):
        _, _, Wr_ee = layer['ee']
        Wl_ea, _, _ = layer['ea']
        _, _, Wr_ae = layer['ae']
        return jnp.concatenate([Wl_ea, Wr_ee + Wr_ae], axis=1)

    def awc(layer):
        Wl_ae, _, _ = layer['ae']
        _, _, Wr_ea = layer['ea']
        return jnp.concatenate([Wl_ae, Wr_ea], axis=1)

    def ebias(layer):
        return (layer['ee'][1] + layer['ae'][1]).reshape(1, _D)

    def abias(layer):
        return layer['ea'][1].reshape(1, _D)

    (yee,) = _tc_proj_cat(x_entity, We, be.reshape(1, _D),
                          convs[0]['ee'][0], [(4, 32)])
    yea, r_e = _tc_proj_cat(x_entity, We, be.reshape(1, _D),
                            ewc2(convs[0]), [(2, 64), (1, 128)])
    yae, r_a = _tc_proj_cat(x_attribute, Wa, ba.reshape(1, _D),
                            awc(convs[0]), _A_SPECS)

    for li in range(2):
        s_ee = _sc_ee(yee, src_ee, dst_ee)
        s_ea, s_ae = _sc_a(yea, yae, src_ea, dst_ea, src_ae, dst_ae)
        if li == 0:
            yee, yea, r_e = _tc_combine_cat(
                s_ee, inv_ee, s_ae, inv_ae, r_e, ebias(convs[0]),
                ewc(convs[1]), None, _E_SPECS)
            yae, r_a = _tc_combine_cat(
                None, None, s_ea, inv_ea, r_a, abias(convs[0]),
                awc(convs[1]), None, _A_SPECS)
        else:
            nba = _NAP // _BN
            (z_hi,) = _tc_combine_cat(
                s_ee, inv_ee, None, None, r_e, ebias(convs[1]),
                Woe, boe.reshape(1, _D), _Z_SPECS, sb=nba)
            (z_lo,) = _tc_combine_cat(
                s_ee, inv_ee, s_ae, inv_ae, r_e, ebias(convs[1]),
                Woe, boe.reshape(1, _D), _Z_SPECS, nblocks=nba)
            z_e = jnp.concatenate([z_lo, z_hi], axis=0)
            (z_a,) = _tc_combine_cat(
                None, None, s_ea, inv_ea, r_a, abias(convs[1]),
                Woa, boa.reshape(1, _D), _Z_SPECS)

    return (z_e[:_NE], z_a[:_NA])
